# Initial kernel scaffold; baseline (speedup 1.0000x reference)
import jax
import jax.numpy as jnp
from jax.experimental import pallas as pl


def kernel(edge_index_u2i, edge_index_i2u, target_ids, emb_user, emb_item, Wl_0_u2i, bl_0_u2i, Wr_0_u2i, Wl_0_i2u, bl_0_i2u, Wr_0_i2u, ln_g_0_user, ln_b_0_user, ln_g_0_item, ln_b_0_item, Wl_1_u2i, bl_1_u2i, Wr_1_u2i, Wl_1_i2u, bl_1_i2u, Wr_1_i2u, ln_g_1_user, ln_b_1_user, ln_g_1_item, ln_b_1_item, head_W1, head_b1, head_W2, head_b2):
    raise NotImplementedError("write your pallas kernel here")


# speedup vs baseline: 4.2164x; 4.2164x over previous
import jax
import jax.numpy as jnp
from jax import lax
from jax.experimental import pallas as pl
from jax.experimental.pallas import tpu as pltpu
from jax.experimental.pallas import tpu_sc as plsc

N = 10000
E = 320000
H = 128
B = 4096

NC, NS = 2, 16
CB = 128
IB = 8
N_PAD = 10240
RPT = N_PAD // NS
PAD_DST = N + 100

EPT0 = 20480
EPT1 = 10240

_mesh = plsc.VectorSubcoreMesh(core_axis_name="c", subcore_axis_name="s")
_sc_params = pltpu.CompilerParams(needs_layout_passes=False)


def _accumulate_relation(src_e, dst_e, x_src, idx_src, idx_dst, rows, acc, sem,
                         tile, n_blocks, cnt_priv):
    ones = jnp.full((16,), 1.0, jnp.float32)

    def block(b, carry):
        pltpu.sync_copy(src_e.at[tile, pl.ds(b * IB, IB)], idx_src)
        pltpu.sync_copy(dst_e.at[tile, pl.ds(b * IB, IB)], idx_dst)
        for r in range(IB):
            pltpu.async_copy(x_src.at[idx_src.at[r]], rows, sem).wait()
            pltpu.sync_copy(rows, acc.at[idx_dst.at[r]], add=True)
            if cnt_priv is not None:
                for k in range(CB // 16):
                    d = idx_dst[r, pl.ds(k * 16, 16)]
                    plsc.addupdate_scatter(cnt_priv, [d], ones)
        return carry

    lax.fori_loop(0, n_blocks, block, 0)


def _sc_layer0(srcA, dstA, srcB, dstB, x_user, x_item, zeros,
               out_item, out_user, out_cnt,
               idx_src, idx_dst, rows, cnt_priv, acc, sem):
    c = lax.axis_index("c")
    s = lax.axis_index("s")
    wid = c * NS + s
    n_blocks = EPT0 // (IB * CB)

    pltpu.sync_copy(zeros.at[pl.ds(s * RPT, RPT)], acc.at[pl.ds(s * RPT, RPT)])
    z16 = jnp.zeros((16,), jnp.float32)

    def zstep(t, carry):
        cnt_priv[pl.ds(t * 16, 16)] = z16
        return carry

    lax.fori_loop(0, N_PAD // 16, zstep, 0)
    plsc.subcore_barrier()

    @pl.when(c == 0)
    def _():
        _accumulate_relation(srcA, dstA, x_user, idx_src, idx_dst, rows, acc,
                             sem, s, n_blocks, cnt_priv)

    @pl.when(c == 1)
    def _():
        _accumulate_relation(srcB, dstB, x_item, idx_src, idx_dst, rows, acc,
                             sem, s, n_blocks, cnt_priv)

    pltpu.sync_copy(cnt_priv, out_cnt.at[pl.ds(wid * N_PAD, N_PAD)])
    plsc.subcore_barrier()

    @pl.when(c == 0)
    def _():
        pltpu.sync_copy(acc.at[pl.ds(s * RPT, RPT)],
                        out_item.at[pl.ds(s * RPT, RPT)])

    @pl.when(c == 1)
    def _():
        pltpu.sync_copy(acc.at[pl.ds(s * RPT, RPT)],
                        out_user.at[pl.ds(s * RPT, RPT)])


def _sc_layer1(src1, dst1, x_user, zeros, out_part,
               idx_src, idx_dst, rows, acc, sem):
    c = lax.axis_index("c")
    s = lax.axis_index("s")
    wid = c * NS + s
    n_blocks = EPT1 // (IB * CB)

    pltpu.sync_copy(zeros.at[pl.ds(s * RPT, RPT)], acc.at[pl.ds(s * RPT, RPT)])
    plsc.subcore_barrier()
    _accumulate_relation(src1, dst1, x_user, idx_src, idx_dst, rows, acc, sem,
                         wid, n_blocks, None)
    plsc.subcore_barrier()
    pltpu.sync_copy(acc.at[pl.ds(s * RPT, RPT)],
                    out_part.at[c, pl.ds(s * RPT, RPT)])


def _sc_gather(table, tids, out, idx_v, rows_v, sem):
    c = lax.axis_index("c")
    s = lax.axis_index("s")
    wid = c * NS + s
    bpw = B // (NC * NS)
    pltpu.sync_copy(tids.at[pl.ds(wid * bpw, bpw)], idx_v)
    pltpu.async_copy(table.at[idx_v], rows_v, sem).wait()
    pltpu.sync_copy(rows_v, out.at[pl.ds(wid * bpw, bpw)])


_seg0 = pl.kernel(
    _sc_layer0, mesh=_mesh, compiler_params=_sc_params,
    out_type=(
        jax.ShapeDtypeStruct((N_PAD, H), jnp.float32),
        jax.ShapeDtypeStruct((N_PAD, H), jnp.float32),
        jax.ShapeDtypeStruct((NC * NS * N_PAD,), jnp.float32),
    ),
    scratch_types=[
        pltpu.VMEM((IB, CB), jnp.int32),
        pltpu.VMEM((IB, CB), jnp.int32),
        pltpu.VMEM((CB, H), jnp.float32),
        pltpu.VMEM((N_PAD,), jnp.float32),
        pltpu.VMEM_SHARED((N_PAD, H), jnp.float32),
        pltpu.SemaphoreType.DMA,
    ],
)

_seg1 = pl.kernel(
    _sc_layer1, mesh=_mesh, compiler_params=_sc_params,
    out_type=jax.ShapeDtypeStruct((NC, N_PAD, H), jnp.float32),
    scratch_types=[
        pltpu.VMEM((IB, CB), jnp.int32),
        pltpu.VMEM((IB, CB), jnp.int32),
        pltpu.VMEM((CB, H), jnp.float32),
        pltpu.VMEM_SHARED((N_PAD, H), jnp.float32),
        pltpu.SemaphoreType.DMA,
    ],
)

_gather = pl.kernel(
    _sc_gather, mesh=_mesh, compiler_params=_sc_params,
    out_type=jax.ShapeDtypeStruct((B, H), jnp.float32),
    scratch_types=[
        pltpu.VMEM((B // (NC * NS),), jnp.int32),
        pltpu.VMEM((B // (NC * NS), H), jnp.float32),
        pltpu.SemaphoreType.DMA,
    ],
)


_RB = 2000


def _dense_body(nparts, refs):
    *sum_refs, cnt_ref, x_ref, wl_ref, bl_ref, wr_ref, g_ref, b_ref, o_ref = refs
    summed = sum_refs[0][...]
    for r in sum_refs[1:]:
        summed = summed + r[...]
    cnt = jnp.sum(cnt_ref[...], axis=1, keepdims=True)
    inv = 1.0 / jnp.maximum(cnt, 1.0)
    mean = summed * inv
    z = (jnp.dot(mean, wl_ref[...], preferred_element_type=jnp.float32)
         + bl_ref[...]
         + jnp.dot(x_ref[...], wr_ref[...], preferred_element_type=jnp.float32))
    mu = jnp.mean(z, axis=-1, keepdims=True)
    zc = z - mu
    var = jnp.mean(zc * zc, axis=-1, keepdims=True)
    y = zc * lax.rsqrt(var + 1e-5) * g_ref[...] + b_ref[...]
    o_ref[...] = jnp.maximum(y, 0.0)


def _dense(summed_list, cnt, x_dst, wl, bl, wr, g, b):
    np_ = len(summed_list)
    row_spec = pl.BlockSpec((_RB, H), lambda i: (i, 0))
    full = pl.BlockSpec((H, H), lambda i: (0, 0))
    vec = pl.BlockSpec((1, H), lambda i: (0, 0))
    return pl.pallas_call(
        lambda *refs: _dense_body(np_, refs),
        grid=(N // _RB,),
        in_specs=[row_spec] * np_ + [
            pl.BlockSpec((_RB, NS), lambda i: (i, 0)),
            row_spec, full, vec, full, vec, vec,
        ],
        out_specs=row_spec,
        out_shape=jax.ShapeDtypeStruct((N, H), jnp.float32),
    )(*summed_list, cnt, x_dst, wl.reshape(H, H), bl.reshape(1, H),
      wr.reshape(H, H), g.reshape(1, H), b.reshape(1, H))


def _head_body(h_ref, w1_ref, b1_ref, w2_ref, b2_ref, o_ref):
    y = jnp.maximum(
        jnp.dot(h_ref[...], w1_ref[...], preferred_element_type=jnp.float32)
        + b1_ref[...], 0.0)
    w2 = w2_ref[...]
    o_ref[...] = jnp.sum(y * w2[:, 0][None, :], axis=1, keepdims=True) + b2_ref[...]


def _head(hrows, w1, b1, w2, b2):
    hb = 1024
    return pl.pallas_call(
        _head_body,
        grid=(B // hb,),
        in_specs=[
            pl.BlockSpec((hb, H), lambda i: (i, 0)),
            pl.BlockSpec((H, H // 2), lambda i: (0, 0)),
            pl.BlockSpec((1, H // 2), lambda i: (0, 0)),
            pl.BlockSpec((H // 2, 1), lambda i: (0, 0)),
            pl.BlockSpec((1, 1), lambda i: (0, 0)),
        ],
        out_specs=pl.BlockSpec((hb, 1), lambda i: (i, 0)),
        out_shape=jax.ShapeDtypeStruct((B, 1), jnp.float32),
    )(hrows, w1, b1.reshape(1, H // 2), w2, b2.reshape(1, 1))


def _pad_edges(ei, n_tiles, ept):
    share = E // n_tiles
    src = ei[0].reshape(n_tiles, share)
    dst = ei[1].reshape(n_tiles, share)
    pad = ept - share
    src = jnp.pad(src, ((0, 0), (0, pad)))
    dst = jnp.pad(dst, ((0, 0), (0, pad)), constant_values=PAD_DST)
    return (src.reshape(n_tiles, ept // CB, CB),
            dst.reshape(n_tiles, ept // CB, CB))


def kernel(edge_index_u2i, edge_index_i2u, target_ids, emb_user, emb_item,
           Wl_0_u2i, bl_0_u2i, Wr_0_u2i, Wl_0_i2u, bl_0_i2u, Wr_0_i2u,
           ln_g_0_user, ln_b_0_user, ln_g_0_item, ln_b_0_item,
           Wl_1_u2i, bl_1_u2i, Wr_1_u2i, Wl_1_i2u, bl_1_i2u, Wr_1_i2u,
           ln_g_1_user, ln_b_1_user, ln_g_1_item, ln_b_1_item,
           head_W1, head_b1, head_W2, head_b2):
    sA, dA = _pad_edges(edge_index_u2i, NS, EPT0)
    sB, dB = _pad_edges(edge_index_i2u, NS, EPT0)
    s1, d1 = _pad_edges(edge_index_u2i, NC * NS, EPT1)

    zeros = jnp.zeros((N_PAD, H), jnp.float32)

    sum_item0, sum_user0, cnt_raw = _seg0(sA, dA, sB, dB,
                                          emb_user, emb_item, zeros)
    cnt_parts = cnt_raw.reshape(NC, NS, N_PAD)
    cnt_item = cnt_parts[0].T
    cnt_user = cnt_parts[1].T

    x1_item = _dense([sum_item0], cnt_item, emb_item,
                     Wl_0_u2i, bl_0_u2i, Wr_0_u2i, ln_g_0_item, ln_b_0_item)
    x1_user = _dense([sum_user0], cnt_user, emb_user,
                     Wl_0_i2u, bl_0_i2u, Wr_0_i2u, ln_g_0_user, ln_b_0_user)

    parts = _seg1(s1, d1, x1_user, zeros)
    x2_item = _dense([parts[0], parts[1]], cnt_item, x1_item,
                     Wl_1_u2i, bl_1_u2i, Wr_1_u2i, ln_g_1_item, ln_b_1_item)

    hrows = _gather(x2_item, target_ids)
    out = _head(hrows, head_W1, head_b1, head_W2, head_b2)
    return out[:, 0]


# speedup vs baseline: 4.8443x; 1.1489x over previous
import jax
import jax.numpy as jnp
from jax import lax
from jax.experimental import pallas as pl
from jax.experimental.pallas import tpu as pltpu
from jax.experimental.pallas import tpu_sc as plsc

N = 10000
E = 320000
H = 128
B = 4096

NC, NS = 2, 16
CB = 128
IB = 8
N_PAD = 10240
RPT = N_PAD // NS
PAD_DST = N + 100

EPT0 = 20480 + 2 * 1024
EPT1 = 10240 + 2 * 1024
NB0 = 20480 // (IB * CB)
NB1 = 10240 // (IB * CB)

_mesh = plsc.VectorSubcoreMesh(core_axis_name="c", subcore_axis_name="s")
_sc_params = pltpu.CompilerParams(needs_layout_passes=False)


def _run_block(x_src, isrc, idst, rows2, gs2, ss2, acc, cnt_priv, ones):
    g = [None, None]
    s = [None, None]
    g[0] = pltpu.async_copy(x_src.at[isrc.at[0]], rows2[0], gs2[0])
    for r in range(IB):
        p = r & 1
        g[p].wait()
        if r + 1 < IB:
            if s[1 - p] is not None:
                s[1 - p].wait()
            g[1 - p] = pltpu.async_copy(x_src.at[isrc.at[r + 1]],
                                        rows2[1 - p], gs2[1 - p])
        s[p] = pltpu.async_copy(rows2[p], acc.at[idst.at[r]], ss2[p], add=True)
        if cnt_priv is not None:
            for k in range(CB // 16):
                d = idst[r, pl.ds(k * 16, 16)]
                plsc.addupdate_scatter(cnt_priv, [d], ones)
    s[0].wait()
    s[1].wait()


def _accumulate_relation(src_e, dst_e, x_src, bufs, acc, tile, n_blocks,
                         cnt_priv):
    (isrc0, idst0, isrc1, idst1, rows0, rows1,
     is0s, is0d, is1s, is1d, gs0, gs1, ss0, ss1) = bufs
    rows2 = (rows0, rows1)
    gs2 = (gs0, gs1)
    ss2 = (ss0, ss1)
    ones = jnp.full((16,), 1.0, jnp.float32)

    def fire(b, sbuf, dbuf, ssem, dsem):
        pltpu.async_copy(src_e.at[tile, pl.ds(b * IB, IB)], sbuf, ssem)
        pltpu.async_copy(dst_e.at[tile, pl.ds(b * IB, IB)], dbuf, dsem)

    def drain(sbuf, dbuf, ssem, dsem):
        pltpu.make_async_copy(src_e.at[tile, pl.ds(0, IB)], sbuf, ssem).wait()
        pltpu.make_async_copy(dst_e.at[tile, pl.ds(0, IB)], dbuf, dsem).wait()

    fire(0, isrc0, idst0, is0s, is0d)
    fire(1, isrc1, idst1, is1s, is1d)

    def pair(k, carry):
        b0 = 2 * k
        drain(isrc0, idst0, is0s, is0d)
        _run_block(x_src, isrc0, idst0, rows2, gs2, ss2, acc, cnt_priv, ones)
        fire(b0 + 2, isrc0, idst0, is0s, is0d)
        drain(isrc1, idst1, is1s, is1d)
        _run_block(x_src, isrc1, idst1, rows2, gs2, ss2, acc, cnt_priv, ones)
        fire(b0 + 3, isrc1, idst1, is1s, is1d)
        return carry

    lax.fori_loop(0, n_blocks // 2, pair, 0)
    drain(isrc0, idst0, is0s, is0d)
    drain(isrc1, idst1, is1s, is1d)


def _sc_layer0(srcA, dstA, srcB, dstB, x_user, x_item, zeros,
               out_item, out_user, out_cnt,
               isrc0, idst0, isrc1, idst1, rows0, rows1, cnt_priv, acc,
               is0s, is0d, is1s, is1d, gs0, gs1, ss0, ss1):
    c = lax.axis_index("c")
    s = lax.axis_index("s")
    wid = c * NS + s
    bufs = (isrc0, idst0, isrc1, idst1, rows0, rows1,
            is0s, is0d, is1s, is1d, gs0, gs1, ss0, ss1)

    pltpu.sync_copy(zeros.at[pl.ds(s * RPT, RPT)], acc.at[pl.ds(s * RPT, RPT)])
    z16 = jnp.zeros((16,), jnp.float32)

    def zstep(t, carry):
        cnt_priv[pl.ds(t * 16, 16)] = z16
        return carry

    lax.fori_loop(0, N_PAD // 16, zstep, 0)
    plsc.subcore_barrier()

    @pl.when(c == 0)
    def _():
        _accumulate_relation(srcA, dstA, x_user, bufs, acc, s, NB0, cnt_priv)

    @pl.when(c == 1)
    def _():
        _accumulate_relation(srcB, dstB, x_item, bufs, acc, s, NB0, cnt_priv)

    pltpu.sync_copy(cnt_priv, out_cnt.at[pl.ds(wid * N_PAD, N_PAD)])
    plsc.subcore_barrier()

    @pl.when(c == 0)
    def _():
        pltpu.sync_copy(acc.at[pl.ds(s * RPT, RPT)],
                        out_item.at[pl.ds(s * RPT, RPT)])

    @pl.when(c == 1)
    def _():
        pltpu.sync_copy(acc.at[pl.ds(s * RPT, RPT)],
                        out_user.at[pl.ds(s * RPT, RPT)])


def _sc_layer1(src1, dst1, x_user, zeros, out_part,
               isrc0, idst0, isrc1, idst1, rows0, rows1, acc,
               is0s, is0d, is1s, is1d, gs0, gs1, ss0, ss1):
    c = lax.axis_index("c")
    s = lax.axis_index("s")
    wid = c * NS + s
    bufs = (isrc0, idst0, isrc1, idst1, rows0, rows1,
            is0s, is0d, is1s, is1d, gs0, gs1, ss0, ss1)

    pltpu.sync_copy(zeros.at[pl.ds(s * RPT, RPT)], acc.at[pl.ds(s * RPT, RPT)])
    plsc.subcore_barrier()
    _accumulate_relation(src1, dst1, x_user, bufs, acc, wid, NB1, None)
    plsc.subcore_barrier()
    pltpu.sync_copy(acc.at[pl.ds(s * RPT, RPT)],
                    out_part.at[c, pl.ds(s * RPT, RPT)])


def _sc_gather(table, tids, out, idx_v, rows_v, sem):
    c = lax.axis_index("c")
    s = lax.axis_index("s")
    wid = c * NS + s
    bpw = B // (NC * NS)
    pltpu.sync_copy(tids.at[pl.ds(wid * bpw, bpw)], idx_v)
    pltpu.async_copy(table.at[idx_v], rows_v, sem).wait()
    pltpu.sync_copy(rows_v, out.at[pl.ds(wid * bpw, bpw)])


_seg0 = pl.kernel(
    _sc_layer0, mesh=_mesh, compiler_params=_sc_params,
    out_type=(
        jax.ShapeDtypeStruct((N_PAD, H), jnp.float32),
        jax.ShapeDtypeStruct((N_PAD, H), jnp.float32),
        jax.ShapeDtypeStruct((NC * NS * N_PAD,), jnp.float32),
    ),
    scratch_types=[
        pltpu.VMEM((IB, CB), jnp.int32),
        pltpu.VMEM((IB, CB), jnp.int32),
        pltpu.VMEM((IB, CB), jnp.int32),
        pltpu.VMEM((IB, CB), jnp.int32),
        pltpu.VMEM((CB, H), jnp.float32),
        pltpu.VMEM((CB, H), jnp.float32),
        pltpu.VMEM((N_PAD,), jnp.float32),
        pltpu.VMEM_SHARED((N_PAD, H), jnp.float32),
    ] + [pltpu.SemaphoreType.DMA] * 8,
)

_seg1 = pl.kernel(
    _sc_layer1, mesh=_mesh, compiler_params=_sc_params,
    out_type=jax.ShapeDtypeStruct((NC, N_PAD, H), jnp.float32),
    scratch_types=[
        pltpu.VMEM((IB, CB), jnp.int32),
        pltpu.VMEM((IB, CB), jnp.int32),
        pltpu.VMEM((IB, CB), jnp.int32),
        pltpu.VMEM((IB, CB), jnp.int32),
        pltpu.VMEM((CB, H), jnp.float32),
        pltpu.VMEM((CB, H), jnp.float32),
        pltpu.VMEM_SHARED((N_PAD, H), jnp.float32),
    ] + [pltpu.SemaphoreType.DMA] * 8,
)

_gather = pl.kernel(
    _sc_gather, mesh=_mesh, compiler_params=_sc_params,
    out_type=jax.ShapeDtypeStruct((B, H), jnp.float32),
    scratch_types=[
        pltpu.VMEM((B // (NC * NS),), jnp.int32),
        pltpu.VMEM((B // (NC * NS), H), jnp.float32),
        pltpu.SemaphoreType.DMA,
    ],
)


_RB = 2000


def _dense_body(nparts, refs):
    *sum_refs, cnt_ref, x_ref, wl_ref, bl_ref, wr_ref, g_ref, b_ref, o_ref = refs
    summed = sum_refs[0][...]
    for r in sum_refs[1:]:
        summed = summed + r[...]
    cnt = jnp.sum(cnt_ref[...], axis=1, keepdims=True)
    inv = 1.0 / jnp.maximum(cnt, 1.0)
    mean = summed * inv
    z = (jnp.dot(mean, wl_ref[...], preferred_element_type=jnp.float32)
         + bl_ref[...]
         + jnp.dot(x_ref[...], wr_ref[...], preferred_element_type=jnp.float32))
    mu = jnp.mean(z, axis=-1, keepdims=True)
    zc = z - mu
    var = jnp.mean(zc * zc, axis=-1, keepdims=True)
    y = zc * lax.rsqrt(var + 1e-5) * g_ref[...] + b_ref[...]
    o_ref[...] = jnp.maximum(y, 0.0)


def _dense(summed_list, cnt, x_dst, wl, bl, wr, g, b):
    np_ = len(summed_list)
    row_spec = pl.BlockSpec((_RB, H), lambda i: (i, 0))
    full = pl.BlockSpec((H, H), lambda i: (0, 0))
    vec = pl.BlockSpec((1, H), lambda i: (0, 0))
    return pl.pallas_call(
        lambda *refs: _dense_body(np_, refs),
        grid=(N // _RB,),
        in_specs=[row_spec] * np_ + [
            pl.BlockSpec((_RB, NS), lambda i: (i, 0)),
            row_spec, full, vec, full, vec, vec,
        ],
        out_specs=row_spec,
        out_shape=jax.ShapeDtypeStruct((N, H), jnp.float32),
    )(*summed_list, cnt, x_dst, wl.reshape(H, H), bl.reshape(1, H),
      wr.reshape(H, H), g.reshape(1, H), b.reshape(1, H))


def _head_body(h_ref, w1_ref, b1_ref, w2_ref, b2_ref, o_ref):
    y = jnp.maximum(
        jnp.dot(h_ref[...], w1_ref[...], preferred_element_type=jnp.float32)
        + b1_ref[...], 0.0)
    w2 = w2_ref[...]
    o_ref[...] = jnp.sum(y * w2[:, 0][None, :], axis=1, keepdims=True) + b2_ref[...]


def _head(hrows, w1, b1, w2, b2):
    hb = 1024
    return pl.pallas_call(
        _head_body,
        grid=(B // hb,),
        in_specs=[
            pl.BlockSpec((hb, H), lambda i: (i, 0)),
            pl.BlockSpec((H, H // 2), lambda i: (0, 0)),
            pl.BlockSpec((1, H // 2), lambda i: (0, 0)),
            pl.BlockSpec((H // 2, 1), lambda i: (0, 0)),
            pl.BlockSpec((1, 1), lambda i: (0, 0)),
        ],
        out_specs=pl.BlockSpec((hb, 1), lambda i: (i, 0)),
        out_shape=jax.ShapeDtypeStruct((B, 1), jnp.float32),
    )(hrows, w1, b1.reshape(1, H // 2), w2, b2.reshape(1, 1))


def _pad_edges(ei, n_tiles, ept):
    share = E // n_tiles
    src = ei[0].reshape(n_tiles, share)
    dst = ei[1].reshape(n_tiles, share)
    pad = ept - share
    src = jnp.pad(src, ((0, 0), (0, pad)))
    dst = jnp.pad(dst, ((0, 0), (0, pad)), constant_values=PAD_DST)
    return (src.reshape(n_tiles, ept // CB, CB),
            dst.reshape(n_tiles, ept // CB, CB))


def kernel(edge_index_u2i, edge_index_i2u, target_ids, emb_user, emb_item,
           Wl_0_u2i, bl_0_u2i, Wr_0_u2i, Wl_0_i2u, bl_0_i2u, Wr_0_i2u,
           ln_g_0_user, ln_b_0_user, ln_g_0_item, ln_b_0_item,
           Wl_1_u2i, bl_1_u2i, Wr_1_u2i, Wl_1_i2u, bl_1_i2u, Wr_1_i2u,
           ln_g_1_user, ln_b_1_user, ln_g_1_item, ln_b_1_item,
           head_W1, head_b1, head_W2, head_b2):
    sA, dA = _pad_edges(edge_index_u2i, NS, EPT0)
    sB, dB = _pad_edges(edge_index_i2u, NS, EPT0)
    s1, d1 = _pad_edges(edge_index_u2i, NC * NS, EPT1)

    zeros = jnp.zeros((N_PAD, H), jnp.float32)

    sum_item0, sum_user0, cnt_raw = _seg0(sA, dA, sB, dB,
                                          emb_user, emb_item, zeros)
    cnt_parts = cnt_raw.reshape(NC, NS, N_PAD)
    cnt_item = cnt_parts[0].T
    cnt_user = cnt_parts[1].T

    x1_item = _dense([sum_item0], cnt_item, emb_item,
                     Wl_0_u2i, bl_0_u2i, Wr_0_u2i, ln_g_0_item, ln_b_0_item)
    x1_user = _dense([sum_user0], cnt_user, emb_user,
                     Wl_0_i2u, bl_0_i2u, Wr_0_i2u, ln_g_0_user, ln_b_0_user)

    parts = _seg1(s1, d1, x1_user, zeros)
    x2_item = _dense([parts[0], parts[1]], cnt_item, x1_item,
                     Wl_1_u2i, bl_1_u2i, Wr_1_u2i, ln_g_1_item, ln_b_1_item)

    hrows = _gather(x2_item, target_ids)
    out = _head(hrows, head_W1, head_b1, head_W2, head_b2)
    return out[:, 0]


# speedup vs baseline: 5.1175x; 1.0564x over previous
import jax
import jax.numpy as jnp
from jax import lax
from jax.experimental import pallas as pl
from jax.experimental.pallas import tpu as pltpu
from jax.experimental.pallas import tpu_sc as plsc

N = 10000
E = 320000
H = 128
B = 4096

NC, NS = 2, 16
CB = 64
IB = 16
DEPTH = 4
N_PAD = 10240
RPT = N_PAD // NS
PAD_DST = N + 100

EPT0 = 20480
EPT1 = 10240
NB0 = EPT0 // (IB * CB)
NB1 = EPT1 // (IB * CB)

_mesh = plsc.VectorSubcoreMesh(core_axis_name="c", subcore_axis_name="s")
_sc_params = pltpu.CompilerParams(needs_layout_passes=False)


def _run_block(b, src_e, dst_e, x_src, isrc, idst, rows, gs, ss, acc,
               cnt_priv, ones, tile):
    pltpu.sync_copy(src_e.at[tile, pl.ds(b * IB, IB)], isrc)
    pltpu.sync_copy(dst_e.at[tile, pl.ds(b * IB, IB)], idst)
    g = [None] * DEPTH
    s = [None] * DEPTH

    def swait_prev(q):
        @pl.when(b > 0)
        def _():
            pltpu.make_async_copy(rows[q], acc.at[idst.at[0]], ss[q]).wait()

    for j in range(DEPTH - 1):
        swait_prev(j)
        g[j] = pltpu.async_copy(x_src.at[isrc.at[j]], rows[j], gs[j])
    for r in range(IB):
        q = r % DEPTH
        g[q].wait()
        nxt = r + DEPTH - 1
        if nxt < IB:
            qn = nxt % DEPTH
            if nxt - DEPTH >= 0:
                s[qn].wait()
            elif nxt == DEPTH - 1:
                swait_prev(qn)
            g[qn] = pltpu.async_copy(x_src.at[isrc.at[nxt]], rows[qn], gs[qn])
        s[q] = pltpu.async_copy(rows[q], acc.at[idst.at[r]], ss[q], add=True)
        if cnt_priv is not None:
            for k in range(CB // 16):
                d = idst[r, pl.ds(k * 16, 16)]
                plsc.addupdate_scatter(cnt_priv, [d], ones)


def _accumulate_relation(src_e, dst_e, x_src, bufs, acc, tile, n_blocks,
                         cnt_priv):
    (isrc, idst, rows0, rows1, rows2, rows3,
     gs0, gs1, gs2, gs3, ss0, ss1, ss2, ss3) = bufs
    rows = (rows0, rows1, rows2, rows3)
    gs = (gs0, gs1, gs2, gs3)
    ss = (ss0, ss1, ss2, ss3)
    ones = jnp.full((16,), 1.0, jnp.float32)

    def block(b, carry):
        _run_block(b, src_e, dst_e, x_src, isrc, idst, rows, gs, ss, acc,
                   cnt_priv, ones, tile)
        return carry

    lax.fori_loop(0, n_blocks, block, 0)
    for q in range(DEPTH):
        pltpu.make_async_copy(rows[q], acc.at[idst.at[0]], ss[q]).wait()


def _sc_layer0(srcA, dstA, srcB, dstB, x_user, x_item, zeros,
               out_item, out_user, out_cnt,
               isrc, idst, rows0, rows1, rows2, rows3, cnt_priv, acc,
               gs0, gs1, gs2, gs3, ss0, ss1, ss2, ss3):
    c = lax.axis_index("c")
    s = lax.axis_index("s")
    wid = c * NS + s
    bufs = (isrc, idst, rows0, rows1, rows2, rows3,
            gs0, gs1, gs2, gs3, ss0, ss1, ss2, ss3)

    pltpu.sync_copy(zeros.at[pl.ds(s * RPT, RPT)], acc.at[pl.ds(s * RPT, RPT)])
    z16 = jnp.zeros((16,), jnp.float32)

    def zstep(t, carry):
        cnt_priv[pl.ds(t * 16, 16)] = z16
        return carry

    lax.fori_loop(0, N_PAD // 16, zstep, 0)
    plsc.subcore_barrier()

    @pl.when(c == 0)
    def _():
        _accumulate_relation(srcA, dstA, x_user, bufs, acc, s, NB0, cnt_priv)

    @pl.when(c == 1)
    def _():
        _accumulate_relation(srcB, dstB, x_item, bufs, acc, s, NB0, cnt_priv)

    pltpu.sync_copy(cnt_priv, out_cnt.at[pl.ds(wid * N_PAD, N_PAD)])
    plsc.subcore_barrier()

    @pl.when(c == 0)
    def _():
        pltpu.sync_copy(acc.at[pl.ds(s * RPT, RPT)],
                        out_item.at[pl.ds(s * RPT, RPT)])

    @pl.when(c == 1)
    def _():
        pltpu.sync_copy(acc.at[pl.ds(s * RPT, RPT)],
                        out_user.at[pl.ds(s * RPT, RPT)])


def _sc_layer1(src1, dst1, x_user, zeros, out_part,
               isrc, idst, rows0, rows1, rows2, rows3, acc,
               gs0, gs1, gs2, gs3, ss0, ss1, ss2, ss3):
    c = lax.axis_index("c")
    s = lax.axis_index("s")
    wid = c * NS + s
    bufs = (isrc, idst, rows0, rows1, rows2, rows3,
            gs0, gs1, gs2, gs3, ss0, ss1, ss2, ss3)

    pltpu.sync_copy(zeros.at[pl.ds(s * RPT, RPT)], acc.at[pl.ds(s * RPT, RPT)])
    plsc.subcore_barrier()
    _accumulate_relation(src1, dst1, x_user, bufs, acc, wid, NB1, None)
    plsc.subcore_barrier()
    pltpu.sync_copy(acc.at[pl.ds(s * RPT, RPT)],
                    out_part.at[c, pl.ds(s * RPT, RPT)])


def _sc_gather(table, tids, out, idx_v, rows_v, sem):
    c = lax.axis_index("c")
    s = lax.axis_index("s")
    wid = c * NS + s
    bpw = B // (NC * NS)
    pltpu.sync_copy(tids.at[pl.ds(wid * bpw, bpw)], idx_v)
    pltpu.async_copy(table.at[idx_v], rows_v, sem).wait()
    pltpu.sync_copy(rows_v, out.at[pl.ds(wid * bpw, bpw)])


_seg0 = pl.kernel(
    _sc_layer0, mesh=_mesh, compiler_params=_sc_params,
    out_type=(
        jax.ShapeDtypeStruct((N_PAD, H), jnp.float32),
        jax.ShapeDtypeStruct((N_PAD, H), jnp.float32),
        jax.ShapeDtypeStruct((NC * NS * N_PAD,), jnp.float32),
    ),
    scratch_types=[
        pltpu.VMEM((IB, CB), jnp.int32),
        pltpu.VMEM((IB, CB), jnp.int32),
        pltpu.VMEM((CB, H), jnp.float32),
        pltpu.VMEM((CB, H), jnp.float32),
        pltpu.VMEM((CB, H), jnp.float32),
        pltpu.VMEM((CB, H), jnp.float32),
        pltpu.VMEM((N_PAD,), jnp.float32),
        pltpu.VMEM_SHARED((N_PAD, H), jnp.float32),
    ] + [pltpu.SemaphoreType.DMA] * 8,
)

_seg1 = pl.kernel(
    _sc_layer1, mesh=_mesh, compiler_params=_sc_params,
    out_type=jax.ShapeDtypeStruct((NC, N_PAD, H), jnp.float32),
    scratch_types=[
        pltpu.VMEM((IB, CB), jnp.int32),
        pltpu.VMEM((IB, CB), jnp.int32),
        pltpu.VMEM((CB, H), jnp.float32),
        pltpu.VMEM((CB, H), jnp.float32),
        pltpu.VMEM((CB, H), jnp.float32),
        pltpu.VMEM((CB, H), jnp.float32),
        pltpu.VMEM_SHARED((N_PAD, H), jnp.float32),
    ] + [pltpu.SemaphoreType.DMA] * 8,
)

_gather = pl.kernel(
    _sc_gather, mesh=_mesh, compiler_params=_sc_params,
    out_type=jax.ShapeDtypeStruct((B, H), jnp.float32),
    scratch_types=[
        pltpu.VMEM((B // (NC * NS),), jnp.int32),
        pltpu.VMEM((B // (NC * NS), H), jnp.float32),
        pltpu.SemaphoreType.DMA,
    ],
)


_RB = 2000


def _dense_body(nparts, refs):
    *sum_refs, cnt_ref, x_ref, wl_ref, bl_ref, wr_ref, g_ref, b_ref, o_ref = refs
    summed = sum_refs[0][...]
    for r in sum_refs[1:]:
        summed = summed + r[...]
    cnt = jnp.sum(cnt_ref[...], axis=1, keepdims=True)
    inv = 1.0 / jnp.maximum(cnt, 1.0)
    mean = summed * inv
    z = (jnp.dot(mean, wl_ref[...], preferred_element_type=jnp.float32)
         + bl_ref[...]
         + jnp.dot(x_ref[...], wr_ref[...], preferred_element_type=jnp.float32))
    mu = jnp.mean(z, axis=-1, keepdims=True)
    zc = z - mu
    var = jnp.mean(zc * zc, axis=-1, keepdims=True)
    y = zc * lax.rsqrt(var + 1e-5) * g_ref[...] + b_ref[...]
    o_ref[...] = jnp.maximum(y, 0.0)


def _dense(summed_list, cnt, x_dst, wl, bl, wr, g, b):
    np_ = len(summed_list)
    row_spec = pl.BlockSpec((_RB, H), lambda i: (i, 0))
    full = pl.BlockSpec((H, H), lambda i: (0, 0))
    vec = pl.BlockSpec((1, H), lambda i: (0, 0))
    return pl.pallas_call(
        lambda *refs: _dense_body(np_, refs),
        grid=(N // _RB,),
        in_specs=[row_spec] * np_ + [
            pl.BlockSpec((_RB, NS), lambda i: (i, 0)),
            row_spec, full, vec, full, vec, vec,
        ],
        out_specs=row_spec,
        out_shape=jax.ShapeDtypeStruct((N, H), jnp.float32),
    )(*summed_list, cnt, x_dst, wl.reshape(H, H), bl.reshape(1, H),
      wr.reshape(H, H), g.reshape(1, H), b.reshape(1, H))


def _head_body(h_ref, w1_ref, b1_ref, w2_ref, b2_ref, o_ref):
    y = jnp.maximum(
        jnp.dot(h_ref[...], w1_ref[...], preferred_element_type=jnp.float32)
        + b1_ref[...], 0.0)
    w2 = w2_ref[...]
    o_ref[...] = jnp.sum(y * w2[:, 0][None, :], axis=1, keepdims=True) + b2_ref[...]


def _head(hrows, w1, b1, w2, b2):
    hb = 1024
    return pl.pallas_call(
        _head_body,
        grid=(B // hb,),
        in_specs=[
            pl.BlockSpec((hb, H), lambda i: (i, 0)),
            pl.BlockSpec((H, H // 2), lambda i: (0, 0)),
            pl.BlockSpec((1, H // 2), lambda i: (0, 0)),
            pl.BlockSpec((H // 2, 1), lambda i: (0, 0)),
            pl.BlockSpec((1, 1), lambda i: (0, 0)),
        ],
        out_specs=pl.BlockSpec((hb, 1), lambda i: (i, 0)),
        out_shape=jax.ShapeDtypeStruct((B, 1), jnp.float32),
    )(hrows, w1, b1.reshape(1, H // 2), w2, b2.reshape(1, 1))


def _pad_edges(ei, n_tiles, ept):
    share = E // n_tiles
    src = ei[0].reshape(n_tiles, share)
    dst = ei[1].reshape(n_tiles, share)
    pad = ept - share
    src = jnp.pad(src, ((0, 0), (0, pad)))
    dst = jnp.pad(dst, ((0, 0), (0, pad)), constant_values=PAD_DST)
    return (src.reshape(n_tiles, ept // CB, CB),
            dst.reshape(n_tiles, ept // CB, CB))


def kernel(edge_index_u2i, edge_index_i2u, target_ids, emb_user, emb_item,
           Wl_0_u2i, bl_0_u2i, Wr_0_u2i, Wl_0_i2u, bl_0_i2u, Wr_0_i2u,
           ln_g_0_user, ln_b_0_user, ln_g_0_item, ln_b_0_item,
           Wl_1_u2i, bl_1_u2i, Wr_1_u2i, Wl_1_i2u, bl_1_i2u, Wr_1_i2u,
           ln_g_1_user, ln_b_1_user, ln_g_1_item, ln_b_1_item,
           head_W1, head_b1, head_W2, head_b2):
    sA, dA = _pad_edges(edge_index_u2i, NS, EPT0)
    sB, dB = _pad_edges(edge_index_i2u, NS, EPT0)
    s1, d1 = _pad_edges(edge_index_u2i, NC * NS, EPT1)

    zeros = jnp.zeros((N_PAD, H), jnp.float32)

    sum_item0, sum_user0, cnt_raw = _seg0(sA, dA, sB, dB,
                                          emb_user, emb_item, zeros)
    cnt_parts = cnt_raw.reshape(NC, NS, N_PAD)
    cnt_item = cnt_parts[0].T
    cnt_user = cnt_parts[1].T

    x1_item = _dense([sum_item0], cnt_item, emb_item,
                     Wl_0_u2i, bl_0_u2i, Wr_0_u2i, ln_g_0_item, ln_b_0_item)
    x1_user = _dense([sum_user0], cnt_user, emb_user,
                     Wl_0_i2u, bl_0_i2u, Wr_0_i2u, ln_g_0_user, ln_b_0_user)

    parts = _seg1(s1, d1, x1_user, zeros)
    x2_item = _dense([parts[0], parts[1]], cnt_item, x1_item,
                     Wl_1_u2i, bl_1_u2i, Wr_1_u2i, ln_g_1_item, ln_b_1_item)

    hrows = _gather(x2_item, target_ids)
    out = _head(hrows, head_W1, head_b1, head_W2, head_b2)
    return out[:, 0]


# speedup vs baseline: 5.9100x; 1.1549x over previous
import jax
import jax.numpy as jnp
from jax import lax
from jax.experimental import pallas as pl
from jax.experimental.pallas import tpu as pltpu
from jax.experimental.pallas import tpu_sc as plsc

N = 10000
E = 320000
H = 128
B = 4096

NC, NS = 2, 16
CB = 64
IB = 16
DEPTH = 4
N_PAD = 10240
RPT = N_PAD // NS
PAD_DST = N + 100

EPT0 = 20480
EPT1 = 10240
NB0 = EPT0 // (IB * CB)
NB1 = EPT1 // (IB * CB)

_mesh = plsc.VectorSubcoreMesh(core_axis_name="c", subcore_axis_name="s")
_sc_params = pltpu.CompilerParams(needs_layout_passes=False)


def _run_block(b, src_e, dst_e, x_src, isrc, idst, rows, gs, ss, acc,
               cnt_priv, ones, tile):
    pltpu.sync_copy(src_e.at[tile, pl.ds(b * IB, IB)], isrc)
    pltpu.sync_copy(dst_e.at[tile, pl.ds(b * IB, IB)], idst)
    g = [None] * DEPTH
    s = [None] * DEPTH

    def swait_prev(q):
        @pl.when(b > 0)
        def _():
            pltpu.make_async_copy(rows[q], acc.at[idst.at[0]], ss[q]).wait()

    for j in range(DEPTH - 1):
        swait_prev(j)
        g[j] = pltpu.async_copy(x_src.at[isrc.at[j]], rows[j], gs[j])
    for r in range(IB):
        q = r % DEPTH
        g[q].wait()
        nxt = r + DEPTH - 1
        if nxt < IB:
            qn = nxt % DEPTH
            if nxt - DEPTH >= 0:
                s[qn].wait()
            elif nxt == DEPTH - 1:
                swait_prev(qn)
            g[qn] = pltpu.async_copy(x_src.at[isrc.at[nxt]], rows[qn], gs[qn])
        s[q] = pltpu.async_copy(rows[q], acc.at[idst.at[r]], ss[q], add=True)
        if cnt_priv is not None:
            for k in range(CB // 16):
                d = idst[r, pl.ds(k * 16, 16)]
                plsc.addupdate_scatter(cnt_priv, [d], ones)


def _accumulate_relation(src_e, dst_e, x_src, bufs, acc, tile, n_blocks,
                         cnt_priv):
    (isrc, idst, rows0, rows1, rows2, rows3,
     gs0, gs1, gs2, gs3, ss0, ss1, ss2, ss3) = bufs
    rows = (rows0, rows1, rows2, rows3)
    gs = (gs0, gs1, gs2, gs3)
    ss = (ss0, ss1, ss2, ss3)
    ones = jnp.full((16,), 1.0, jnp.float32)

    def block(b, carry):
        _run_block(b, src_e, dst_e, x_src, isrc, idst, rows, gs, ss, acc,
                   cnt_priv, ones, tile)
        return carry

    lax.fori_loop(0, n_blocks, block, 0)
    for q in range(DEPTH):
        pltpu.make_async_copy(rows[q], acc.at[idst.at[0]], ss[q]).wait()


def _sc_layer0(srcA, dstA, srcB, dstB, x_user, x_item, zeros,
               out_item, out_user, out_cnt,
               isrc, idst, rows0, rows1, rows2, rows3, cnt_priv, acc,
               gs0, gs1, gs2, gs3, ss0, ss1, ss2, ss3):
    c = lax.axis_index("c")
    s = lax.axis_index("s")
    wid = c * NS + s
    bufs = (isrc, idst, rows0, rows1, rows2, rows3,
            gs0, gs1, gs2, gs3, ss0, ss1, ss2, ss3)

    pltpu.sync_copy(zeros.at[pl.ds(s * RPT, RPT)], acc.at[pl.ds(s * RPT, RPT)])
    z16 = jnp.zeros((16,), jnp.float32)

    def zstep(t, carry):
        cnt_priv[pl.ds(t * 16, 16)] = z16
        return carry

    lax.fori_loop(0, N_PAD // 16, zstep, 0)
    plsc.subcore_barrier()

    @pl.when(c == 0)
    def _():
        _accumulate_relation(srcA, dstA, x_user, bufs, acc, s, NB0, cnt_priv)

    @pl.when(c == 1)
    def _():
        _accumulate_relation(srcB, dstB, x_item, bufs, acc, s, NB0, cnt_priv)

    pltpu.sync_copy(cnt_priv, out_cnt.at[pl.ds(wid * N_PAD, N_PAD)])
    plsc.subcore_barrier()

    @pl.when(c == 0)
    def _():
        pltpu.sync_copy(acc.at[pl.ds(s * RPT, RPT)],
                        out_item.at[pl.ds(s * RPT, RPT)])

    @pl.when(c == 1)
    def _():
        pltpu.sync_copy(acc.at[pl.ds(s * RPT, RPT)],
                        out_user.at[pl.ds(s * RPT, RPT)])


def _sc_layer1(src1, dst1, x_user, zeros, out_part,
               isrc, idst, rows0, rows1, rows2, rows3, acc,
               gs0, gs1, gs2, gs3, ss0, ss1, ss2, ss3):
    c = lax.axis_index("c")
    s = lax.axis_index("s")
    wid = c * NS + s
    bufs = (isrc, idst, rows0, rows1, rows2, rows3,
            gs0, gs1, gs2, gs3, ss0, ss1, ss2, ss3)

    pltpu.sync_copy(zeros.at[pl.ds(s * RPT, RPT)], acc.at[pl.ds(s * RPT, RPT)])
    plsc.subcore_barrier()
    _accumulate_relation(src1, dst1, x_user.at[c], bufs, acc, wid, NB1, None)
    plsc.subcore_barrier()
    pltpu.sync_copy(acc.at[pl.ds(s * RPT, RPT)],
                    out_part.at[c, pl.ds(s * RPT, RPT)])


def _sc_gather(table, tids, out, idx_v, rows_v, sem):
    c = lax.axis_index("c")
    s = lax.axis_index("s")
    wid = c * NS + s
    bpw = B // (NC * NS)
    pltpu.sync_copy(tids.at[pl.ds(wid * bpw, bpw)], idx_v)
    pltpu.async_copy(table.at[idx_v], rows_v, sem).wait()
    pltpu.sync_copy(rows_v, out.at[pl.ds(wid * bpw, bpw)])


_seg0 = pl.kernel(
    _sc_layer0, mesh=_mesh, compiler_params=_sc_params,
    out_type=(
        jax.ShapeDtypeStruct((N_PAD, H), jnp.float32),
        jax.ShapeDtypeStruct((N_PAD, H), jnp.float32),
        jax.ShapeDtypeStruct((NC * NS * N_PAD,), jnp.float32),
    ),
    scratch_types=[
        pltpu.VMEM((IB, CB), jnp.int32),
        pltpu.VMEM((IB, CB), jnp.int32),
        pltpu.VMEM((CB, H), jnp.float32),
        pltpu.VMEM((CB, H), jnp.float32),
        pltpu.VMEM((CB, H), jnp.float32),
        pltpu.VMEM((CB, H), jnp.float32),
        pltpu.VMEM((N_PAD,), jnp.float32),
        pltpu.VMEM_SHARED((N_PAD, H), jnp.float32),
    ] + [pltpu.SemaphoreType.DMA] * 8,
)

_seg1 = pl.kernel(
    _sc_layer1, mesh=_mesh, compiler_params=_sc_params,
    out_type=jax.ShapeDtypeStruct((NC, N_PAD, H), jnp.float32),
    scratch_types=[
        pltpu.VMEM((IB, CB), jnp.int32),
        pltpu.VMEM((IB, CB), jnp.int32),
        pltpu.VMEM((CB, H), jnp.float32),
        pltpu.VMEM((CB, H), jnp.float32),
        pltpu.VMEM((CB, H), jnp.float32),
        pltpu.VMEM((CB, H), jnp.float32),
        pltpu.VMEM_SHARED((N_PAD, H), jnp.float32),
    ] + [pltpu.SemaphoreType.DMA] * 8,
)

_gather = pl.kernel(
    _sc_gather, mesh=_mesh, compiler_params=_sc_params,
    out_type=jax.ShapeDtypeStruct((B, H), jnp.float32),
    scratch_types=[
        pltpu.VMEM((B // (NC * NS),), jnp.int32),
        pltpu.VMEM((B // (NC * NS), H), jnp.float32),
        pltpu.SemaphoreType.DMA,
    ],
)


_RB = 2000


def _dense_body(nparts, ncopies, refs):
    *sum_refs, cnt_ref, x_ref, wl_ref, bl_ref, wr_ref, g_ref, b_ref, o_ref = refs
    summed = sum_refs[0][...]
    for r in sum_refs[1:]:
        summed = summed + r[...]
    cnt = jnp.sum(cnt_ref[...], axis=1, keepdims=True)
    inv = 1.0 / jnp.maximum(cnt, 1.0)
    mean = summed * inv
    z = (jnp.dot(mean, wl_ref[...], preferred_element_type=jnp.float32)
         + bl_ref[...]
         + jnp.dot(x_ref[...], wr_ref[...], preferred_element_type=jnp.float32))
    mu = jnp.mean(z, axis=-1, keepdims=True)
    zc = z - mu
    var = jnp.mean(zc * zc, axis=-1, keepdims=True)
    y = zc * lax.rsqrt(var + 1e-5) * g_ref[...] + b_ref[...]
    out = jnp.maximum(y, 0.0)
    if ncopies == 1:
        o_ref[...] = out
    else:
        o_ref[...] = jnp.broadcast_to(out[None], (ncopies,) + out.shape)


def _dense(summed_list, cnt, x_dst, wl, bl, wr, g, b, ncopies=1):
    np_ = len(summed_list)
    row_spec = pl.BlockSpec((_RB, H), lambda i: (i, 0))
    full = pl.BlockSpec((H, H), lambda i: (0, 0))
    vec = pl.BlockSpec((1, H), lambda i: (0, 0))
    if ncopies == 1:
        out_spec = row_spec
        out_shape = jax.ShapeDtypeStruct((N, H), jnp.float32)
    else:
        out_spec = pl.BlockSpec((ncopies, _RB, H), lambda i: (0, i, 0))
        out_shape = jax.ShapeDtypeStruct((ncopies, N, H), jnp.float32)
    return pl.pallas_call(
        lambda *refs: _dense_body(np_, ncopies, refs),
        grid=(N // _RB,),
        in_specs=[row_spec] * np_ + [
            pl.BlockSpec((_RB, NS), lambda i: (i, 0)),
            row_spec, full, vec, full, vec, vec,
        ],
        out_specs=out_spec,
        out_shape=out_shape,
    )(*summed_list, cnt, x_dst, wl.reshape(H, H), bl.reshape(1, H),
      wr.reshape(H, H), g.reshape(1, H), b.reshape(1, H))


def _head_body(h_ref, w1_ref, b1_ref, w2_ref, b2_ref, o_ref):
    y = jnp.maximum(
        jnp.dot(h_ref[...], w1_ref[...], preferred_element_type=jnp.float32)
        + b1_ref[...], 0.0)
    w2 = w2_ref[...]
    o_ref[...] = jnp.sum(y * w2[:, 0][None, :], axis=1, keepdims=True) + b2_ref[...]


def _head(hrows, w1, b1, w2, b2):
    hb = 1024
    return pl.pallas_call(
        _head_body,
        grid=(B // hb,),
        in_specs=[
            pl.BlockSpec((hb, H), lambda i: (i, 0)),
            pl.BlockSpec((H, H // 2), lambda i: (0, 0)),
            pl.BlockSpec((1, H // 2), lambda i: (0, 0)),
            pl.BlockSpec((H // 2, 1), lambda i: (0, 0)),
            pl.BlockSpec((1, 1), lambda i: (0, 0)),
        ],
        out_specs=pl.BlockSpec((hb, 1), lambda i: (i, 0)),
        out_shape=jax.ShapeDtypeStruct((B, 1), jnp.float32),
    )(hrows, w1, b1.reshape(1, H // 2), w2, b2.reshape(1, 1))


def _pad_edges(ei, n_tiles, ept):
    share = E // n_tiles
    src = ei[0].reshape(n_tiles, share)
    dst = ei[1].reshape(n_tiles, share)
    pad = ept - share
    src = jnp.pad(src, ((0, 0), (0, pad)))
    dst = jnp.pad(dst, ((0, 0), (0, pad)), constant_values=PAD_DST)
    return (src.reshape(n_tiles, ept // CB, CB),
            dst.reshape(n_tiles, ept // CB, CB))


def kernel(edge_index_u2i, edge_index_i2u, target_ids, emb_user, emb_item,
           Wl_0_u2i, bl_0_u2i, Wr_0_u2i, Wl_0_i2u, bl_0_i2u, Wr_0_i2u,
           ln_g_0_user, ln_b_0_user, ln_g_0_item, ln_b_0_item,
           Wl_1_u2i, bl_1_u2i, Wr_1_u2i, Wl_1_i2u, bl_1_i2u, Wr_1_i2u,
           ln_g_1_user, ln_b_1_user, ln_g_1_item, ln_b_1_item,
           head_W1, head_b1, head_W2, head_b2):
    sA, dA = _pad_edges(edge_index_u2i, NS, EPT0)
    sB, dB = _pad_edges(edge_index_i2u, NS, EPT0)
    s1, d1 = _pad_edges(edge_index_u2i, NC * NS, EPT1)

    zeros = jnp.zeros((N_PAD, H), jnp.float32)

    sum_item0, sum_user0, cnt_raw = _seg0(sA, dA, sB, dB,
                                          emb_user, emb_item, zeros)
    cnt_parts = cnt_raw.reshape(NC, NS, N_PAD)
    cnt_item = cnt_parts[0].T
    cnt_user = cnt_parts[1].T

    x1_item = _dense([sum_item0], cnt_item, emb_item,
                     Wl_0_u2i, bl_0_u2i, Wr_0_u2i, ln_g_0_item, ln_b_0_item)
    x1_user = _dense([sum_user0], cnt_user, emb_user,
                     Wl_0_i2u, bl_0_i2u, Wr_0_i2u, ln_g_0_user, ln_b_0_user,
                     ncopies=NC)

    parts = _seg1(s1, d1, x1_user, zeros)
    x2_item = _dense([parts[0], parts[1]], cnt_item, x1_item,
                     Wl_1_u2i, bl_1_u2i, Wr_1_u2i, ln_g_1_item, ln_b_1_item)

    hrows = _gather(x2_item, target_ids)
    out = _head(hrows, head_W1, head_b1, head_W2, head_b2)
    return out[:, 0]


# speedup vs baseline: 7.5087x; 1.2705x over previous
import jax
import jax.numpy as jnp
from jax import lax
from jax.experimental import pallas as pl
from jax.experimental.pallas import tpu as pltpu
from jax.experimental.pallas import tpu_sc as plsc

N = 10000
E = 320000
H = 128
B = 4096

NC, NS = 2, 16
CB = 64
IB = 16
DEPTH = 4
N_PAD = 10240
RPT = N_PAD // NS
PAD_DST = N + 100

EPT0 = 20480
EPT1 = 10240
NB0 = EPT0 // (IB * CB)
NB1 = EPT1 // (IB * CB)

_mesh = plsc.VectorSubcoreMesh(core_axis_name="c", subcore_axis_name="s")
_sc_params = pltpu.CompilerParams(needs_layout_passes=False)


def _run_block(b, src_e, dst_e, x_src, isrc, idst, rows, gs, ss, acc,
               cnt_priv, ones, tile):
    pltpu.sync_copy(src_e.at[tile, pl.ds(b * IB, IB)], isrc)
    pltpu.sync_copy(dst_e.at[tile, pl.ds(b * IB, IB)], idst)
    g = [None] * DEPTH
    s = [None] * DEPTH

    def swait_prev(q):
        @pl.when(b > 0)
        def _():
            pltpu.make_async_copy(rows[q], acc.at[idst.at[0]], ss[q]).wait()

    for j in range(DEPTH - 1):
        swait_prev(j)
        g[j] = pltpu.async_copy(x_src.at[isrc.at[j]], rows[j], gs[j])
    for r in range(IB):
        q = r % DEPTH
        g[q].wait()
        nxt = r + DEPTH - 1
        if nxt < IB:
            qn = nxt % DEPTH
            if nxt - DEPTH >= 0:
                s[qn].wait()
            elif nxt == DEPTH - 1:
                swait_prev(qn)
            g[qn] = pltpu.async_copy(x_src.at[isrc.at[nxt]], rows[qn], gs[qn])
        s[q] = pltpu.async_copy(rows[q], acc.at[idst.at[r]], ss[q], add=True)
        if cnt_priv is not None:
            for k in range(CB // 16):
                d = idst[r, pl.ds(k * 16, 16)]
                plsc.addupdate_scatter(cnt_priv, [d], ones)


def _accumulate_relation(src_e, dst_e, x_src, bufs, acc, tile, n_blocks,
                         cnt_priv):
    (isrc, idst, rows0, rows1, rows2, rows3,
     gs0, gs1, gs2, gs3, ss0, ss1, ss2, ss3) = bufs
    rows = (rows0, rows1, rows2, rows3)
    gs = (gs0, gs1, gs2, gs3)
    ss = (ss0, ss1, ss2, ss3)
    ones = jnp.full((16,), 1.0, jnp.float32)

    def block(b, carry):
        _run_block(b, src_e, dst_e, x_src, isrc, idst, rows, gs, ss, acc,
                   cnt_priv, ones, tile)
        return carry

    lax.fori_loop(0, n_blocks, block, 0)
    for q in range(DEPTH):
        pltpu.make_async_copy(rows[q], acc.at[idst.at[0]], ss[q]).wait()


def _sc_layer0(srcA, dstA, srcB, dstB, x_user, x_item, zeros,
               out_item, out_user, out_cnt,
               isrc, idst, rows0, rows1, rows2, rows3, cnt_priv, acc,
               gs0, gs1, gs2, gs3, ss0, ss1, ss2, ss3):
    c = lax.axis_index("c")
    s = lax.axis_index("s")
    wid = c * NS + s
    bufs = (isrc, idst, rows0, rows1, rows2, rows3,
            gs0, gs1, gs2, gs3, ss0, ss1, ss2, ss3)

    pltpu.sync_copy(zeros.at[pl.ds(s * RPT, RPT)], acc.at[pl.ds(s * RPT, RPT)])
    z16 = jnp.zeros((16,), jnp.float32)

    def zstep(t, carry):
        cnt_priv[pl.ds(t * 16, 16)] = z16
        return carry

    lax.fori_loop(0, N_PAD // 16, zstep, 0)
    plsc.subcore_barrier()

    cp = s & 1

    @pl.when(c == 0)
    def _():
        _accumulate_relation(srcA, dstA, x_user.at[cp], bufs, acc, s, NB0,
                             cnt_priv)

    @pl.when(c == 1)
    def _():
        _accumulate_relation(srcB, dstB, x_item.at[cp], bufs, acc, s, NB0,
                             cnt_priv)

    pltpu.sync_copy(cnt_priv, out_cnt.at[pl.ds(wid * N_PAD, N_PAD)])
    plsc.subcore_barrier()

    @pl.when(c == 0)
    def _():
        pltpu.sync_copy(acc.at[pl.ds(s * RPT, RPT)],
                        out_item.at[pl.ds(s * RPT, RPT)])

    @pl.when(c == 1)
    def _():
        pltpu.sync_copy(acc.at[pl.ds(s * RPT, RPT)],
                        out_user.at[pl.ds(s * RPT, RPT)])


def _sc_layer1(src1, dst1, x_user, zeros, out_part,
               isrc, idst, rows0, rows1, rows2, rows3, acc,
               gs0, gs1, gs2, gs3, ss0, ss1, ss2, ss3):
    c = lax.axis_index("c")
    s = lax.axis_index("s")
    wid = c * NS + s
    bufs = (isrc, idst, rows0, rows1, rows2, rows3,
            gs0, gs1, gs2, gs3, ss0, ss1, ss2, ss3)

    pltpu.sync_copy(zeros.at[pl.ds(s * RPT, RPT)], acc.at[pl.ds(s * RPT, RPT)])
    plsc.subcore_barrier()
    _accumulate_relation(src1, dst1, x_user.at[c * 2 + (s & 1)], bufs, acc,
                         wid, NB1, None)
    plsc.subcore_barrier()
    pltpu.sync_copy(acc.at[pl.ds(s * RPT, RPT)],
                    out_part.at[c, pl.ds(s * RPT, RPT)])


def _sc_gather(table, tids, out, idx_v, rows_v, sem):
    c = lax.axis_index("c")
    s = lax.axis_index("s")
    wid = c * NS + s
    bpw = B // (NC * NS)
    pltpu.sync_copy(tids.at[pl.ds(wid * bpw, bpw)], idx_v)
    pltpu.async_copy(table.at[idx_v], rows_v, sem).wait()
    pltpu.sync_copy(rows_v, out.at[pl.ds(wid * bpw, bpw)])


_seg0 = pl.kernel(
    _sc_layer0, mesh=_mesh, compiler_params=_sc_params,
    out_type=(
        jax.ShapeDtypeStruct((N_PAD, H), jnp.float32),
        jax.ShapeDtypeStruct((N_PAD, H), jnp.float32),
        jax.ShapeDtypeStruct((NC * NS * N_PAD,), jnp.float32),
    ),
    scratch_types=[
        pltpu.VMEM((IB, CB), jnp.int32),
        pltpu.VMEM((IB, CB), jnp.int32),
        pltpu.VMEM((CB, H), jnp.float32),
        pltpu.VMEM((CB, H), jnp.float32),
        pltpu.VMEM((CB, H), jnp.float32),
        pltpu.VMEM((CB, H), jnp.float32),
        pltpu.VMEM((N_PAD,), jnp.float32),
        pltpu.VMEM_SHARED((N_PAD, H), jnp.float32),
    ] + [pltpu.SemaphoreType.DMA] * 8,
)

_seg1 = pl.kernel(
    _sc_layer1, mesh=_mesh, compiler_params=_sc_params,
    out_type=jax.ShapeDtypeStruct((NC, N_PAD, H), jnp.float32),
    scratch_types=[
        pltpu.VMEM((IB, CB), jnp.int32),
        pltpu.VMEM((IB, CB), jnp.int32),
        pltpu.VMEM((CB, H), jnp.float32),
        pltpu.VMEM((CB, H), jnp.float32),
        pltpu.VMEM((CB, H), jnp.float32),
        pltpu.VMEM((CB, H), jnp.float32),
        pltpu.VMEM_SHARED((N_PAD, H), jnp.float32),
    ] + [pltpu.SemaphoreType.DMA] * 8,
)

_gather = pl.kernel(
    _sc_gather, mesh=_mesh, compiler_params=_sc_params,
    out_type=jax.ShapeDtypeStruct((B, H), jnp.float32),
    scratch_types=[
        pltpu.VMEM((B // (NC * NS),), jnp.int32),
        pltpu.VMEM((B // (NC * NS), H), jnp.float32),
        pltpu.SemaphoreType.DMA,
    ],
)


_RB = 2000


def _dense_body(nparts, ncopies, refs):
    *sum_refs, cnt_ref, x_ref, wl_ref, bl_ref, wr_ref, g_ref, b_ref, o_ref = refs
    summed = sum_refs[0][...]
    for r in sum_refs[1:]:
        summed = summed + r[...]
    cnt = jnp.sum(cnt_ref[...], axis=1, keepdims=True)
    inv = 1.0 / jnp.maximum(cnt, 1.0)
    mean = summed * inv
    z = (jnp.dot(mean, wl_ref[...], preferred_element_type=jnp.float32)
         + bl_ref[...]
         + jnp.dot(x_ref[...], wr_ref[...], preferred_element_type=jnp.float32))
    mu = jnp.mean(z, axis=-1, keepdims=True)
    zc = z - mu
    var = jnp.mean(zc * zc, axis=-1, keepdims=True)
    y = zc * lax.rsqrt(var + 1e-5) * g_ref[...] + b_ref[...]
    out = jnp.maximum(y, 0.0)
    if ncopies == 1:
        o_ref[...] = out
    else:
        o_ref[...] = jnp.broadcast_to(out[None], (ncopies,) + out.shape)


def _dense(summed_list, cnt, x_dst, wl, bl, wr, g, b, ncopies=1):
    np_ = len(summed_list)
    row_spec = pl.BlockSpec((_RB, H), lambda i: (i, 0))
    full = pl.BlockSpec((H, H), lambda i: (0, 0))
    vec = pl.BlockSpec((1, H), lambda i: (0, 0))
    if ncopies == 1:
        out_spec = row_spec
        out_shape = jax.ShapeDtypeStruct((N, H), jnp.float32)
    else:
        out_spec = pl.BlockSpec((ncopies, _RB, H), lambda i: (0, i, 0))
        out_shape = jax.ShapeDtypeStruct((ncopies, N, H), jnp.float32)
    return pl.pallas_call(
        lambda *refs: _dense_body(np_, ncopies, refs),
        grid=(N // _RB,),
        in_specs=[row_spec] * np_ + [
            pl.BlockSpec((_RB, NS), lambda i: (i, 0)),
            row_spec, full, vec, full, vec, vec,
        ],
        out_specs=out_spec,
        out_shape=out_shape,
    )(*summed_list, cnt, x_dst, wl.reshape(H, H), bl.reshape(1, H),
      wr.reshape(H, H), g.reshape(1, H), b.reshape(1, H))


def _head_body(h_ref, w1_ref, b1_ref, w2_ref, b2_ref, o_ref):
    y = jnp.maximum(
        jnp.dot(h_ref[...], w1_ref[...], preferred_element_type=jnp.float32)
        + b1_ref[...], 0.0)
    w2 = w2_ref[...]
    o_ref[...] = jnp.sum(y * w2[:, 0][None, :], axis=1, keepdims=True) + b2_ref[...]


def _head(hrows, w1, b1, w2, b2):
    hb = 1024
    return pl.pallas_call(
        _head_body,
        grid=(B // hb,),
        in_specs=[
            pl.BlockSpec((hb, H), lambda i: (i, 0)),
            pl.BlockSpec((H, H // 2), lambda i: (0, 0)),
            pl.BlockSpec((1, H // 2), lambda i: (0, 0)),
            pl.BlockSpec((H // 2, 1), lambda i: (0, 0)),
            pl.BlockSpec((1, 1), lambda i: (0, 0)),
        ],
        out_specs=pl.BlockSpec((hb, 1), lambda i: (i, 0)),
        out_shape=jax.ShapeDtypeStruct((B, 1), jnp.float32),
    )(hrows, w1, b1.reshape(1, H // 2), w2, b2.reshape(1, 1))


def _pad_edges(ei, n_tiles, ept):
    share = E // n_tiles
    src = ei[0].reshape(n_tiles, share)
    dst = ei[1].reshape(n_tiles, share)
    pad = ept - share
    src = jnp.pad(src, ((0, 0), (0, pad)))
    dst = jnp.pad(dst, ((0, 0), (0, pad)), constant_values=PAD_DST)
    return (src.reshape(n_tiles, ept // CB, CB),
            dst.reshape(n_tiles, ept // CB, CB))


def kernel(edge_index_u2i, edge_index_i2u, target_ids, emb_user, emb_item,
           Wl_0_u2i, bl_0_u2i, Wr_0_u2i, Wl_0_i2u, bl_0_i2u, Wr_0_i2u,
           ln_g_0_user, ln_b_0_user, ln_g_0_item, ln_b_0_item,
           Wl_1_u2i, bl_1_u2i, Wr_1_u2i, Wl_1_i2u, bl_1_i2u, Wr_1_i2u,
           ln_g_1_user, ln_b_1_user, ln_g_1_item, ln_b_1_item,
           head_W1, head_b1, head_W2, head_b2):
    sA, dA = _pad_edges(edge_index_u2i, NS, EPT0)
    sB, dB = _pad_edges(edge_index_i2u, NS, EPT0)
    s1, d1 = _pad_edges(edge_index_u2i, NC * NS, EPT1)

    zeros = jnp.zeros((N_PAD, H), jnp.float32)
    emb_user2 = jnp.broadcast_to(emb_user[None], (2, N, H))
    emb_item2 = jnp.broadcast_to(emb_item[None], (2, N, H))

    sum_item0, sum_user0, cnt_raw = _seg0(sA, dA, sB, dB,
                                          emb_user2, emb_item2, zeros)
    cnt_parts = cnt_raw.reshape(NC, NS, N_PAD)
    cnt_item = cnt_parts[0].T
    cnt_user = cnt_parts[1].T

    x1_item = _dense([sum_item0], cnt_item, emb_item,
                     Wl_0_u2i, bl_0_u2i, Wr_0_u2i, ln_g_0_item, ln_b_0_item)
    x1_user = _dense([sum_user0], cnt_user, emb_user,
                     Wl_0_i2u, bl_0_i2u, Wr_0_i2u, ln_g_0_user, ln_b_0_user,
                     ncopies=2 * NC)

    parts = _seg1(s1, d1, x1_user, zeros)
    x2_item = _dense([parts[0], parts[1]], cnt_item, x1_item,
                     Wl_1_u2i, bl_1_u2i, Wr_1_u2i, ln_g_1_item, ln_b_1_item)

    hrows = _gather(x2_item, target_ids)
    out = _head(hrows, head_W1, head_b1, head_W2, head_b2)
    return out[:, 0]


# speedup vs baseline: 8.1984x; 1.0919x over previous
import jax
import jax.numpy as jnp
from jax import lax
from jax.experimental import pallas as pl
from jax.experimental.pallas import tpu as pltpu
from jax.experimental.pallas import tpu_sc as plsc

N = 10000
E = 320000
H = 128
B = 4096

NC, NS = 2, 16
CB = 64
IB = 16
DEPTH = 4
N_PAD = 10240
RPT = N_PAD // NS
PAD_DST = N + 100

EPT0 = 20480
EPT1 = 10240
NB0 = EPT0 // (IB * CB)
NB1 = EPT1 // (IB * CB)

_mesh = plsc.VectorSubcoreMesh(core_axis_name="c", subcore_axis_name="s")
_sc_params = pltpu.CompilerParams(needs_layout_passes=False)


def _run_block(b, src_e, dst_e, x_src, isrc, idst, rows, gs, ss, acc,
               cnt_priv, ones, tile):
    pltpu.sync_copy(src_e.at[tile, pl.ds(b * IB, IB)], isrc)
    pltpu.sync_copy(dst_e.at[tile, pl.ds(b * IB, IB)], idst)
    g = [None] * DEPTH
    s = [None] * DEPTH

    def swait_prev(q):
        @pl.when(b > 0)
        def _():
            pltpu.make_async_copy(rows[q], acc.at[idst.at[0]], ss[q]).wait()

    for j in range(DEPTH - 1):
        swait_prev(j)
        g[j] = pltpu.async_copy(x_src.at[isrc.at[j]], rows[j], gs[j])
    for r in range(IB):
        q = r % DEPTH
        g[q].wait()
        nxt = r + DEPTH - 1
        if nxt < IB:
            qn = nxt % DEPTH
            if nxt - DEPTH >= 0:
                s[qn].wait()
            elif nxt == DEPTH - 1:
                swait_prev(qn)
            g[qn] = pltpu.async_copy(x_src.at[isrc.at[nxt]], rows[qn], gs[qn])
        s[q] = pltpu.async_copy(rows[q], acc.at[idst.at[r]], ss[q], add=True)
        if cnt_priv is not None:
            for k in range(CB // 16):
                d = idst[r, pl.ds(k * 16, 16)]
                plsc.addupdate_scatter(cnt_priv, [d], ones)


def _accumulate_relation(src_e, dst_e, x_src, bufs, acc, tile, n_blocks,
                         cnt_priv):
    (isrc, idst, rows0, rows1, rows2, rows3,
     gs0, gs1, gs2, gs3, ss0, ss1, ss2, ss3) = bufs
    rows = (rows0, rows1, rows2, rows3)
    gs = (gs0, gs1, gs2, gs3)
    ss = (ss0, ss1, ss2, ss3)
    ones = jnp.full((16,), 1.0, jnp.float32)

    def block(b, carry):
        _run_block(b, src_e, dst_e, x_src, isrc, idst, rows, gs, ss, acc,
                   cnt_priv, ones, tile)
        return carry

    lax.fori_loop(0, n_blocks, block, 0)
    for q in range(DEPTH):
        pltpu.make_async_copy(rows[q], acc.at[idst.at[0]], ss[q]).wait()


def _sc_layer0(srcA, dstA, srcB, dstB, x_user, x_item, zeros,
               out_item, out_user, out_cnt,
               isrc, idst, rows0, rows1, rows2, rows3, cnt_priv, acc,
               gs0, gs1, gs2, gs3, ss0, ss1, ss2, ss3):
    c = lax.axis_index("c")
    s = lax.axis_index("s")
    wid = c * NS + s
    bufs = (isrc, idst, rows0, rows1, rows2, rows3,
            gs0, gs1, gs2, gs3, ss0, ss1, ss2, ss3)

    pltpu.sync_copy(zeros.at[pl.ds(s * RPT, RPT)], acc.at[pl.ds(s * RPT, RPT)])
    z16 = jnp.zeros((16,), jnp.float32)

    def zstep(t, carry):
        cnt_priv[pl.ds(t * 16, 16)] = z16
        return carry

    lax.fori_loop(0, N_PAD // 16, zstep, 0)
    plsc.subcore_barrier()

    cp = s & 3

    @pl.when(c == 0)
    def _():
        _accumulate_relation(srcA, dstA, x_user.at[cp], bufs, acc, s, NB0,
                             cnt_priv)

    @pl.when(c == 1)
    def _():
        _accumulate_relation(srcB, dstB, x_item.at[cp], bufs, acc, s, NB0,
                             cnt_priv)

    pltpu.sync_copy(cnt_priv, out_cnt.at[pl.ds(wid * N_PAD, N_PAD)])
    plsc.subcore_barrier()

    @pl.when(c == 0)
    def _():
        pltpu.sync_copy(acc.at[pl.ds(s * RPT, RPT)],
                        out_item.at[pl.ds(s * RPT, RPT)])

    @pl.when(c == 1)
    def _():
        pltpu.sync_copy(acc.at[pl.ds(s * RPT, RPT)],
                        out_user.at[pl.ds(s * RPT, RPT)])


def _sc_layer1(src1, dst1, x_user, zeros, out_part,
               isrc, idst, rows0, rows1, rows2, rows3, acc,
               gs0, gs1, gs2, gs3, ss0, ss1, ss2, ss3):
    c = lax.axis_index("c")
    s = lax.axis_index("s")
    wid = c * NS + s
    bufs = (isrc, idst, rows0, rows1, rows2, rows3,
            gs0, gs1, gs2, gs3, ss0, ss1, ss2, ss3)

    pltpu.sync_copy(zeros.at[pl.ds(s * RPT, RPT)], acc.at[pl.ds(s * RPT, RPT)])
    plsc.subcore_barrier()
    _accumulate_relation(src1, dst1, x_user.at[c * 2 + (s & 1)], bufs, acc,
                         wid, NB1, None)
    plsc.subcore_barrier()
    pltpu.sync_copy(acc.at[pl.ds(s * RPT, RPT)],
                    out_part.at[c, pl.ds(s * RPT, RPT)])


def _sc_gather(table, tids, out, idx_v, rows_v, sem):
    c = lax.axis_index("c")
    s = lax.axis_index("s")
    wid = c * NS + s
    bpw = B // (NC * NS)
    pltpu.sync_copy(tids.at[pl.ds(wid * bpw, bpw)], idx_v)
    pltpu.async_copy(table.at[idx_v], rows_v, sem).wait()
    pltpu.sync_copy(rows_v, out.at[pl.ds(wid * bpw, bpw)])


_seg0 = pl.kernel(
    _sc_layer0, mesh=_mesh, compiler_params=_sc_params,
    out_type=(
        jax.ShapeDtypeStruct((N_PAD, H), jnp.float32),
        jax.ShapeDtypeStruct((N_PAD, H), jnp.float32),
        jax.ShapeDtypeStruct((NC * NS * N_PAD,), jnp.float32),
    ),
    scratch_types=[
        pltpu.VMEM((IB, CB), jnp.int32),
        pltpu.VMEM((IB, CB), jnp.int32),
        pltpu.VMEM((CB, H), jnp.float32),
        pltpu.VMEM((CB, H), jnp.float32),
        pltpu.VMEM((CB, H), jnp.float32),
        pltpu.VMEM((CB, H), jnp.float32),
        pltpu.VMEM((N_PAD,), jnp.float32),
        pltpu.VMEM_SHARED((N_PAD, H), jnp.float32),
    ] + [pltpu.SemaphoreType.DMA] * 8,
)

_seg1 = pl.kernel(
    _sc_layer1, mesh=_mesh, compiler_params=_sc_params,
    out_type=jax.ShapeDtypeStruct((NC, N_PAD, H), jnp.float32),
    scratch_types=[
        pltpu.VMEM((IB, CB), jnp.int32),
        pltpu.VMEM((IB, CB), jnp.int32),
        pltpu.VMEM((CB, H), jnp.float32),
        pltpu.VMEM((CB, H), jnp.float32),
        pltpu.VMEM((CB, H), jnp.float32),
        pltpu.VMEM((CB, H), jnp.float32),
        pltpu.VMEM_SHARED((N_PAD, H), jnp.float32),
    ] + [pltpu.SemaphoreType.DMA] * 8,
)

_gather = pl.kernel(
    _sc_gather, mesh=_mesh, compiler_params=_sc_params,
    out_type=jax.ShapeDtypeStruct((B, H), jnp.float32),
    scratch_types=[
        pltpu.VMEM((B // (NC * NS),), jnp.int32),
        pltpu.VMEM((B // (NC * NS), H), jnp.float32),
        pltpu.SemaphoreType.DMA,
    ],
)


_RB = 2000


def _dense_body(nparts, ncopies, refs):
    *sum_refs, cnt_ref, x_ref, wl_ref, bl_ref, wr_ref, g_ref, b_ref, o_ref = refs
    summed = sum_refs[0][...]
    for r in sum_refs[1:]:
        summed = summed + r[...]
    cnt = jnp.sum(cnt_ref[...], axis=1, keepdims=True)
    inv = 1.0 / jnp.maximum(cnt, 1.0)
    mean = summed * inv
    z = (jnp.dot(mean, wl_ref[...], preferred_element_type=jnp.float32)
         + bl_ref[...]
         + jnp.dot(x_ref[...], wr_ref[...], preferred_element_type=jnp.float32))
    mu = jnp.mean(z, axis=-1, keepdims=True)
    zc = z - mu
    var = jnp.mean(zc * zc, axis=-1, keepdims=True)
    y = zc * lax.rsqrt(var + 1e-5) * g_ref[...] + b_ref[...]
    out = jnp.maximum(y, 0.0)
    if ncopies == 1:
        o_ref[...] = out
    else:
        o_ref[...] = jnp.broadcast_to(out[None], (ncopies,) + out.shape)


def _dense(summed_list, cnt, x_dst, wl, bl, wr, g, b, ncopies=1):
    np_ = len(summed_list)
    row_spec = pl.BlockSpec((_RB, H), lambda i: (i, 0))
    full = pl.BlockSpec((H, H), lambda i: (0, 0))
    vec = pl.BlockSpec((1, H), lambda i: (0, 0))
    if ncopies == 1:
        out_spec = row_spec
        out_shape = jax.ShapeDtypeStruct((N, H), jnp.float32)
    else:
        out_spec = pl.BlockSpec((ncopies, _RB, H), lambda i: (0, i, 0))
        out_shape = jax.ShapeDtypeStruct((ncopies, N, H), jnp.float32)
    return pl.pallas_call(
        lambda *refs: _dense_body(np_, ncopies, refs),
        grid=(N // _RB,),
        in_specs=[row_spec] * np_ + [
            pl.BlockSpec((_RB, NS), lambda i: (i, 0)),
            row_spec, full, vec, full, vec, vec,
        ],
        out_specs=out_spec,
        out_shape=out_shape,
    )(*summed_list, cnt, x_dst, wl.reshape(H, H), bl.reshape(1, H),
      wr.reshape(H, H), g.reshape(1, H), b.reshape(1, H))


def _head_body(h_ref, w1_ref, b1_ref, w2_ref, b2_ref, o_ref):
    y = jnp.maximum(
        jnp.dot(h_ref[...], w1_ref[...], preferred_element_type=jnp.float32)
        + b1_ref[...], 0.0)
    w2 = w2_ref[...]
    o_ref[...] = jnp.sum(y * w2[:, 0][None, :], axis=1, keepdims=True) + b2_ref[...]


def _head(hrows, w1, b1, w2, b2):
    hb = 1024
    return pl.pallas_call(
        _head_body,
        grid=(B // hb,),
        in_specs=[
            pl.BlockSpec((hb, H), lambda i: (i, 0)),
            pl.BlockSpec((H, H // 2), lambda i: (0, 0)),
            pl.BlockSpec((1, H // 2), lambda i: (0, 0)),
            pl.BlockSpec((H // 2, 1), lambda i: (0, 0)),
            pl.BlockSpec((1, 1), lambda i: (0, 0)),
        ],
        out_specs=pl.BlockSpec((hb, 1), lambda i: (i, 0)),
        out_shape=jax.ShapeDtypeStruct((B, 1), jnp.float32),
    )(hrows, w1, b1.reshape(1, H // 2), w2, b2.reshape(1, 1))


def _pad_edges(ei, n_tiles, ept):
    share = E // n_tiles
    src = ei[0].reshape(n_tiles, share)
    dst = ei[1].reshape(n_tiles, share)
    pad = ept - share
    src = jnp.pad(src, ((0, 0), (0, pad)))
    dst = jnp.pad(dst, ((0, 0), (0, pad)), constant_values=PAD_DST)
    return (src.reshape(n_tiles, ept // CB, CB),
            dst.reshape(n_tiles, ept // CB, CB))


def kernel(edge_index_u2i, edge_index_i2u, target_ids, emb_user, emb_item,
           Wl_0_u2i, bl_0_u2i, Wr_0_u2i, Wl_0_i2u, bl_0_i2u, Wr_0_i2u,
           ln_g_0_user, ln_b_0_user, ln_g_0_item, ln_b_0_item,
           Wl_1_u2i, bl_1_u2i, Wr_1_u2i, Wl_1_i2u, bl_1_i2u, Wr_1_i2u,
           ln_g_1_user, ln_b_1_user, ln_g_1_item, ln_b_1_item,
           head_W1, head_b1, head_W2, head_b2):
    sA, dA = _pad_edges(edge_index_u2i, NS, EPT0)
    sB, dB = _pad_edges(edge_index_i2u, NS, EPT0)
    s1, d1 = _pad_edges(edge_index_u2i, NC * NS, EPT1)

    zeros = jnp.zeros((N_PAD, H), jnp.float32)
    emb_user2 = jnp.broadcast_to(emb_user[None], (4, N, H))
    emb_item2 = jnp.broadcast_to(emb_item[None], (4, N, H))

    sum_item0, sum_user0, cnt_raw = _seg0(sA, dA, sB, dB,
                                          emb_user2, emb_item2, zeros)
    cnt_parts = cnt_raw.reshape(NC, NS, N_PAD)
    cnt_item = cnt_parts[0].T
    cnt_user = cnt_parts[1].T

    x1_item = _dense([sum_item0], cnt_item, emb_item,
                     Wl_0_u2i, bl_0_u2i, Wr_0_u2i, ln_g_0_item, ln_b_0_item)
    x1_user = _dense([sum_user0], cnt_user, emb_user,
                     Wl_0_i2u, bl_0_i2u, Wr_0_i2u, ln_g_0_user, ln_b_0_user,
                     ncopies=2 * NC)

    parts = _seg1(s1, d1, x1_user, zeros)
    x2_item = _dense([parts[0], parts[1]], cnt_item, x1_item,
                     Wl_1_u2i, bl_1_u2i, Wr_1_u2i, ln_g_1_item, ln_b_1_item)

    hrows = _gather(x2_item, target_ids)
    out = _head(hrows, head_W1, head_b1, head_W2, head_b2)
    return out[:, 0]


# speedup vs baseline: 8.2345x; 1.0044x over previous
import jax
import jax.numpy as jnp
from jax import lax
from jax.experimental import pallas as pl
from jax.experimental.pallas import tpu as pltpu
from jax.experimental.pallas import tpu_sc as plsc

N = 10000
E = 320000
H = 128
B = 4096

NC, NS = 2, 16
CB = 64
IB = 16
DEPTH = 4
N_PAD = 10240
RPT = N_PAD // NS
PAD_DST = N + 100

EPT0 = 20480
EPT1 = 10240
NB0 = EPT0 // (IB * CB)
NB1 = EPT1 // (IB * CB)

_mesh = plsc.VectorSubcoreMesh(core_axis_name="c", subcore_axis_name="s")
_sc_params = pltpu.CompilerParams(needs_layout_passes=False)


def _run_block(b, src_e, dst_e, x_src, isrc, idst, rows, gs, ss, acc,
               cnt_priv, ones, tile):
    pltpu.sync_copy(src_e.at[tile, pl.ds(b * IB, IB)], isrc)
    pltpu.sync_copy(dst_e.at[tile, pl.ds(b * IB, IB)], idst)
    g = [None] * DEPTH
    s = [None] * DEPTH

    def swait_prev(q):
        @pl.when(b > 0)
        def _():
            pltpu.make_async_copy(rows[q], acc.at[idst.at[0]], ss[q]).wait()

    for j in range(DEPTH - 1):
        swait_prev(j)
        g[j] = pltpu.async_copy(x_src.at[isrc.at[j]], rows[j], gs[j])
    for r in range(IB):
        q = r % DEPTH
        g[q].wait()
        nxt = r + DEPTH - 1
        if nxt < IB:
            qn = nxt % DEPTH
            if nxt - DEPTH >= 0:
                s[qn].wait()
            elif nxt == DEPTH - 1:
                swait_prev(qn)
            g[qn] = pltpu.async_copy(x_src.at[isrc.at[nxt]], rows[qn], gs[qn])
        s[q] = pltpu.async_copy(rows[q], acc.at[idst.at[r]], ss[q], add=True)
        if cnt_priv is not None:
            for k in range(CB // 16):
                d = idst[r, pl.ds(k * 16, 16)]
                plsc.addupdate_scatter(cnt_priv, [d], ones)


def _accumulate_relation(src_e, dst_e, x_src, bufs, acc, tile, n_blocks,
                         cnt_priv):
    (isrc, idst, rows0, rows1, rows2, rows3,
     gs0, gs1, gs2, gs3, ss0, ss1, ss2, ss3) = bufs
    rows = (rows0, rows1, rows2, rows3)
    gs = (gs0, gs1, gs2, gs3)
    ss = (ss0, ss1, ss2, ss3)
    ones = jnp.full((16,), 1.0, jnp.float32)

    def block(b, carry):
        _run_block(b, src_e, dst_e, x_src, isrc, idst, rows, gs, ss, acc,
                   cnt_priv, ones, tile)
        return carry

    lax.fori_loop(0, n_blocks, block, 0)
    for q in range(DEPTH):
        pltpu.make_async_copy(rows[q], acc.at[idst.at[0]], ss[q]).wait()


def _sc_layer0(srcA, dstA, srcB, dstB, x_user, x_item, zeros,
               out_item, out_user, out_cnt,
               isrc, idst, rows0, rows1, rows2, rows3, cnt_priv, acc,
               gs0, gs1, gs2, gs3, ss0, ss1, ss2, ss3):
    c = lax.axis_index("c")
    s = lax.axis_index("s")
    wid = c * NS + s
    bufs = (isrc, idst, rows0, rows1, rows2, rows3,
            gs0, gs1, gs2, gs3, ss0, ss1, ss2, ss3)

    pltpu.sync_copy(zeros.at[pl.ds(s * RPT, RPT)], acc.at[pl.ds(s * RPT, RPT)])
    z16 = jnp.zeros((16,), jnp.float32)

    def zstep(t, carry):
        cnt_priv[pl.ds(t * 16, 16)] = z16
        return carry

    lax.fori_loop(0, N_PAD // 16, zstep, 0)
    plsc.subcore_barrier()

    cp = s & 7

    @pl.when(c == 0)
    def _():
        _accumulate_relation(srcA, dstA, x_user.at[cp], bufs, acc, s, NB0,
                             cnt_priv)

    @pl.when(c == 1)
    def _():
        _accumulate_relation(srcB, dstB, x_item.at[cp], bufs, acc, s, NB0,
                             cnt_priv)

    pltpu.sync_copy(cnt_priv, out_cnt.at[pl.ds(wid * N_PAD, N_PAD)])
    plsc.subcore_barrier()

    @pl.when(c == 0)
    def _():
        pltpu.sync_copy(acc.at[pl.ds(s * RPT, RPT)],
                        out_item.at[pl.ds(s * RPT, RPT)])

    @pl.when(c == 1)
    def _():
        pltpu.sync_copy(acc.at[pl.ds(s * RPT, RPT)],
                        out_user.at[pl.ds(s * RPT, RPT)])


def _sc_layer1(src1, dst1, x_user, zeros, out_part,
               isrc, idst, rows0, rows1, rows2, rows3, acc,
               gs0, gs1, gs2, gs3, ss0, ss1, ss2, ss3):
    c = lax.axis_index("c")
    s = lax.axis_index("s")
    wid = c * NS + s
    bufs = (isrc, idst, rows0, rows1, rows2, rows3,
            gs0, gs1, gs2, gs3, ss0, ss1, ss2, ss3)

    pltpu.sync_copy(zeros.at[pl.ds(s * RPT, RPT)], acc.at[pl.ds(s * RPT, RPT)])
    plsc.subcore_barrier()
    _accumulate_relation(src1, dst1, x_user.at[c * 2 + (s & 1)], bufs, acc,
                         wid, NB1, None)
    plsc.subcore_barrier()
    pltpu.sync_copy(acc.at[pl.ds(s * RPT, RPT)],
                    out_part.at[c, pl.ds(s * RPT, RPT)])


def _sc_gather(table, tids, out, idx_v, rows_v, sem):
    c = lax.axis_index("c")
    s = lax.axis_index("s")
    wid = c * NS + s
    bpw = B // (NC * NS)
    pltpu.sync_copy(tids.at[pl.ds(wid * bpw, bpw)], idx_v)
    pltpu.async_copy(table.at[idx_v], rows_v, sem).wait()
    pltpu.sync_copy(rows_v, out.at[pl.ds(wid * bpw, bpw)])


_seg0 = pl.kernel(
    _sc_layer0, mesh=_mesh, compiler_params=_sc_params,
    out_type=(
        jax.ShapeDtypeStruct((N_PAD, H), jnp.float32),
        jax.ShapeDtypeStruct((N_PAD, H), jnp.float32),
        jax.ShapeDtypeStruct((NC * NS * N_PAD,), jnp.float32),
    ),
    scratch_types=[
        pltpu.VMEM((IB, CB), jnp.int32),
        pltpu.VMEM((IB, CB), jnp.int32),
        pltpu.VMEM((CB, H), jnp.float32),
        pltpu.VMEM((CB, H), jnp.float32),
        pltpu.VMEM((CB, H), jnp.float32),
        pltpu.VMEM((CB, H), jnp.float32),
        pltpu.VMEM((N_PAD,), jnp.float32),
        pltpu.VMEM_SHARED((N_PAD, H), jnp.float32),
    ] + [pltpu.SemaphoreType.DMA] * 8,
)

_seg1 = pl.kernel(
    _sc_layer1, mesh=_mesh, compiler_params=_sc_params,
    out_type=jax.ShapeDtypeStruct((NC, N_PAD, H), jnp.float32),
    scratch_types=[
        pltpu.VMEM((IB, CB), jnp.int32),
        pltpu.VMEM((IB, CB), jnp.int32),
        pltpu.VMEM((CB, H), jnp.float32),
        pltpu.VMEM((CB, H), jnp.float32),
        pltpu.VMEM((CB, H), jnp.float32),
        pltpu.VMEM((CB, H), jnp.float32),
        pltpu.VMEM_SHARED((N_PAD, H), jnp.float32),
    ] + [pltpu.SemaphoreType.DMA] * 8,
)

_gather = pl.kernel(
    _sc_gather, mesh=_mesh, compiler_params=_sc_params,
    out_type=jax.ShapeDtypeStruct((B, H), jnp.float32),
    scratch_types=[
        pltpu.VMEM((B // (NC * NS),), jnp.int32),
        pltpu.VMEM((B // (NC * NS), H), jnp.float32),
        pltpu.SemaphoreType.DMA,
    ],
)


_RB = 2000


def _dense_body(nparts, ncopies, refs):
    *sum_refs, cnt_ref, x_ref, wl_ref, bl_ref, wr_ref, g_ref, b_ref, o_ref = refs
    summed = sum_refs[0][...]
    for r in sum_refs[1:]:
        summed = summed + r[...]
    cnt = jnp.sum(cnt_ref[...], axis=1, keepdims=True)
    inv = 1.0 / jnp.maximum(cnt, 1.0)
    mean = summed * inv
    z = (jnp.dot(mean, wl_ref[...], preferred_element_type=jnp.float32)
         + bl_ref[...]
         + jnp.dot(x_ref[...], wr_ref[...], preferred_element_type=jnp.float32))
    mu = jnp.mean(z, axis=-1, keepdims=True)
    zc = z - mu
    var = jnp.mean(zc * zc, axis=-1, keepdims=True)
    y = zc * lax.rsqrt(var + 1e-5) * g_ref[...] + b_ref[...]
    out = jnp.maximum(y, 0.0)
    if ncopies == 1:
        o_ref[...] = out
    else:
        o_ref[...] = jnp.broadcast_to(out[None], (ncopies,) + out.shape)


def _dense(summed_list, cnt, x_dst, wl, bl, wr, g, b, ncopies=1):
    np_ = len(summed_list)
    row_spec = pl.BlockSpec((_RB, H), lambda i: (i, 0))
    full = pl.BlockSpec((H, H), lambda i: (0, 0))
    vec = pl.BlockSpec((1, H), lambda i: (0, 0))
    if ncopies == 1:
        out_spec = row_spec
        out_shape = jax.ShapeDtypeStruct((N, H), jnp.float32)
    else:
        out_spec = pl.BlockSpec((ncopies, _RB, H), lambda i: (0, i, 0))
        out_shape = jax.ShapeDtypeStruct((ncopies, N, H), jnp.float32)
    return pl.pallas_call(
        lambda *refs: _dense_body(np_, ncopies, refs),
        grid=(N // _RB,),
        in_specs=[row_spec] * np_ + [
            pl.BlockSpec((_RB, NS), lambda i: (i, 0)),
            row_spec, full, vec, full, vec, vec,
        ],
        out_specs=out_spec,
        out_shape=out_shape,
    )(*summed_list, cnt, x_dst, wl.reshape(H, H), bl.reshape(1, H),
      wr.reshape(H, H), g.reshape(1, H), b.reshape(1, H))


def _head_body(h_ref, w1_ref, b1_ref, w2_ref, b2_ref, o_ref):
    y = jnp.maximum(
        jnp.dot(h_ref[...], w1_ref[...], preferred_element_type=jnp.float32)
        + b1_ref[...], 0.0)
    w2 = w2_ref[...]
    o_ref[...] = jnp.sum(y * w2[:, 0][None, :], axis=1, keepdims=True) + b2_ref[...]


def _head(hrows, w1, b1, w2, b2):
    hb = 1024
    return pl.pallas_call(
        _head_body,
        grid=(B // hb,),
        in_specs=[
            pl.BlockSpec((hb, H), lambda i: (i, 0)),
            pl.BlockSpec((H, H // 2), lambda i: (0, 0)),
            pl.BlockSpec((1, H // 2), lambda i: (0, 0)),
            pl.BlockSpec((H // 2, 1), lambda i: (0, 0)),
            pl.BlockSpec((1, 1), lambda i: (0, 0)),
        ],
        out_specs=pl.BlockSpec((hb, 1), lambda i: (i, 0)),
        out_shape=jax.ShapeDtypeStruct((B, 1), jnp.float32),
    )(hrows, w1, b1.reshape(1, H // 2), w2, b2.reshape(1, 1))


def _pad_edges(ei, n_tiles, ept):
    share = E // n_tiles
    src = ei[0].reshape(n_tiles, share)
    dst = ei[1].reshape(n_tiles, share)
    pad = ept - share
    src = jnp.pad(src, ((0, 0), (0, pad)))
    dst = jnp.pad(dst, ((0, 0), (0, pad)), constant_values=PAD_DST)
    return (src.reshape(n_tiles, ept // CB, CB),
            dst.reshape(n_tiles, ept // CB, CB))


def kernel(edge_index_u2i, edge_index_i2u, target_ids, emb_user, emb_item,
           Wl_0_u2i, bl_0_u2i, Wr_0_u2i, Wl_0_i2u, bl_0_i2u, Wr_0_i2u,
           ln_g_0_user, ln_b_0_user, ln_g_0_item, ln_b_0_item,
           Wl_1_u2i, bl_1_u2i, Wr_1_u2i, Wl_1_i2u, bl_1_i2u, Wr_1_i2u,
           ln_g_1_user, ln_b_1_user, ln_g_1_item, ln_b_1_item,
           head_W1, head_b1, head_W2, head_b2):
    sA, dA = _pad_edges(edge_index_u2i, NS, EPT0)
    sB, dB = _pad_edges(edge_index_i2u, NS, EPT0)
    s1, d1 = _pad_edges(edge_index_u2i, NC * NS, EPT1)

    zeros = jnp.zeros((N_PAD, H), jnp.float32)
    emb_user2 = jnp.broadcast_to(emb_user[None], (8, N, H))
    emb_item2 = jnp.broadcast_to(emb_item[None], (8, N, H))

    sum_item0, sum_user0, cnt_raw = _seg0(sA, dA, sB, dB,
                                          emb_user2, emb_item2, zeros)
    cnt_parts = cnt_raw.reshape(NC, NS, N_PAD)
    cnt_item = cnt_parts[0].T
    cnt_user = cnt_parts[1].T

    x1_item = _dense([sum_item0], cnt_item, emb_item,
                     Wl_0_u2i, bl_0_u2i, Wr_0_u2i, ln_g_0_item, ln_b_0_item)
    x1_user = _dense([sum_user0], cnt_user, emb_user,
                     Wl_0_i2u, bl_0_i2u, Wr_0_i2u, ln_g_0_user, ln_b_0_user,
                     ncopies=2 * NC)

    parts = _seg1(s1, d1, x1_user, zeros)
    x2_item = _dense([parts[0], parts[1]], cnt_item, x1_item,
                     Wl_1_u2i, bl_1_u2i, Wr_1_u2i, ln_g_1_item, ln_b_1_item)

    hrows = _gather(x2_item, target_ids)
    out = _head(hrows, head_W1, head_b1, head_W2, head_b2)
    return out[:, 0]


# speedup vs baseline: 8.5720x; 1.0410x over previous
import jax
import jax.numpy as jnp
from jax import lax
from jax.experimental import pallas as pl
from jax.experimental.pallas import tpu as pltpu
from jax.experimental.pallas import tpu_sc as plsc

N = 10000
E = 320000
H = 128
B = 4096

NC, NS = 2, 16
CB = 64
IB = 16
DEPTH = 4
N_PAD = 10240
RPT = N_PAD // NS
PAD_DST = N + 100

EPT0 = 20480
EPT1 = 10240
NB0 = EPT0 // (IB * CB)
NB1 = EPT1 // (IB * CB)

_mesh = plsc.VectorSubcoreMesh(core_axis_name="c", subcore_axis_name="s")
_sc_params = pltpu.CompilerParams(needs_layout_passes=False)


def _run_block(b, src_e, dst_e, x_src, isrc, idst, rows, gs, ss, acc,
               cnt_priv, ones, tile):
    pltpu.sync_copy(src_e.at[tile, pl.ds(b * IB, IB)], isrc)
    pltpu.sync_copy(dst_e.at[tile, pl.ds(b * IB, IB)], idst)
    g = [None] * DEPTH
    s = [None] * DEPTH

    def swait_prev(q):
        @pl.when(b > 0)
        def _():
            pltpu.make_async_copy(rows[q], acc.at[idst.at[0]], ss[q]).wait()

    for j in range(DEPTH - 1):
        swait_prev(j)
        g[j] = pltpu.async_copy(x_src.at[isrc.at[j]], rows[j], gs[j])
    for r in range(IB):
        q = r % DEPTH
        g[q].wait()
        nxt = r + DEPTH - 1
        if nxt < IB:
            qn = nxt % DEPTH
            if nxt - DEPTH >= 0:
                s[qn].wait()
            elif nxt == DEPTH - 1:
                swait_prev(qn)
            g[qn] = pltpu.async_copy(x_src.at[isrc.at[nxt]], rows[qn], gs[qn])
        s[q] = pltpu.async_copy(rows[q], acc.at[idst.at[r]], ss[q], add=True)
        if cnt_priv is not None:
            for k in range(CB // 16):
                d = idst[r, pl.ds(k * 16, 16)]
                plsc.addupdate_scatter(cnt_priv, [d], ones)


def _accumulate_relation(src_e, dst_e, x_src, bufs, acc, tile, n_blocks,
                         cnt_priv):
    (isrc, idst, rows0, rows1, rows2, rows3,
     gs0, gs1, gs2, gs3, ss0, ss1, ss2, ss3) = bufs
    rows = (rows0, rows1, rows2, rows3)
    gs = (gs0, gs1, gs2, gs3)
    ss = (ss0, ss1, ss2, ss3)
    ones = jnp.full((16,), 1.0, jnp.float32)

    def block(b, carry):
        _run_block(b, src_e, dst_e, x_src, isrc, idst, rows, gs, ss, acc,
                   cnt_priv, ones, tile)
        return carry

    lax.fori_loop(0, n_blocks, block, 0)
    for q in range(DEPTH):
        pltpu.make_async_copy(rows[q], acc.at[idst.at[0]], ss[q]).wait()


def _sc_layer0(srcA, dstA, srcB, dstB, x_user, x_item, zeros,
               out_item, out_user, out_cnt,
               isrc, idst, rows0, rows1, rows2, rows3, cnt_priv, acc,
               gs0, gs1, gs2, gs3, ss0, ss1, ss2, ss3):
    c = lax.axis_index("c")
    s = lax.axis_index("s")
    wid = c * NS + s
    bufs = (isrc, idst, rows0, rows1, rows2, rows3,
            gs0, gs1, gs2, gs3, ss0, ss1, ss2, ss3)

    pltpu.sync_copy(zeros.at[pl.ds(s * RPT, RPT)], acc.at[pl.ds(s * RPT, RPT)])
    z16 = jnp.zeros((16,), jnp.float32)

    def zstep(t, carry):
        cnt_priv[pl.ds(t * 16, 16)] = z16
        return carry

    lax.fori_loop(0, N_PAD // 16, zstep, 0)
    plsc.subcore_barrier()

    cp = s & 3

    @pl.when(c == 0)
    def _():
        _accumulate_relation(srcA, dstA, x_user.at[cp], bufs, acc, s, NB0,
                             cnt_priv)

    @pl.when(c == 1)
    def _():
        _accumulate_relation(srcB, dstB, x_item.at[cp], bufs, acc, s, NB0,
                             cnt_priv)

    pltpu.sync_copy(cnt_priv, out_cnt.at[pl.ds(wid * N_PAD, N_PAD)])
    plsc.subcore_barrier()

    @pl.when(c == 0)
    def _():
        pltpu.sync_copy(acc.at[pl.ds(s * RPT, RPT)],
                        out_item.at[pl.ds(s * RPT, RPT)])

    @pl.when(c == 1)
    def _():
        pltpu.sync_copy(acc.at[pl.ds(s * RPT, RPT)],
                        out_user.at[pl.ds(s * RPT, RPT)])


def _sc_layer1(src1, dst1, x_user, zeros, out_part,
               isrc, idst, rows0, rows1, rows2, rows3, acc,
               gs0, gs1, gs2, gs3, ss0, ss1, ss2, ss3):
    c = lax.axis_index("c")
    s = lax.axis_index("s")
    wid = c * NS + s
    bufs = (isrc, idst, rows0, rows1, rows2, rows3,
            gs0, gs1, gs2, gs3, ss0, ss1, ss2, ss3)

    pltpu.sync_copy(zeros.at[pl.ds(s * RPT, RPT)], acc.at[pl.ds(s * RPT, RPT)])
    plsc.subcore_barrier()
    _accumulate_relation(src1, dst1, x_user.at[c * 4 + (s & 3)], bufs, acc,
                         wid, NB1, None)
    plsc.subcore_barrier()
    pltpu.sync_copy(acc.at[pl.ds(s * RPT, RPT)],
                    out_part.at[c, pl.ds(s * RPT, RPT)])


def _sc_gather(table, tids, out, idx_v, rows_v, sem):
    c = lax.axis_index("c")
    s = lax.axis_index("s")
    wid = c * NS + s
    bpw = B // (NC * NS)
    pltpu.sync_copy(tids.at[pl.ds(wid * bpw, bpw)], idx_v)
    pltpu.async_copy(table.at[idx_v], rows_v, sem).wait()
    pltpu.sync_copy(rows_v, out.at[pl.ds(wid * bpw, bpw)])


_seg0 = pl.kernel(
    _sc_layer0, mesh=_mesh, compiler_params=_sc_params,
    out_type=(
        jax.ShapeDtypeStruct((N_PAD, H), jnp.float32),
        jax.ShapeDtypeStruct((N_PAD, H), jnp.float32),
        jax.ShapeDtypeStruct((NC * NS * N_PAD,), jnp.float32),
    ),
    scratch_types=[
        pltpu.VMEM((IB, CB), jnp.int32),
        pltpu.VMEM((IB, CB), jnp.int32),
        pltpu.VMEM((CB, H), jnp.float32),
        pltpu.VMEM((CB, H), jnp.float32),
        pltpu.VMEM((CB, H), jnp.float32),
        pltpu.VMEM((CB, H), jnp.float32),
        pltpu.VMEM((N_PAD,), jnp.float32),
        pltpu.VMEM_SHARED((N_PAD, H), jnp.float32),
    ] + [pltpu.SemaphoreType.DMA] * 8,
)

_seg1 = pl.kernel(
    _sc_layer1, mesh=_mesh, compiler_params=_sc_params,
    out_type=jax.ShapeDtypeStruct((NC, N_PAD, H), jnp.float32),
    scratch_types=[
        pltpu.VMEM((IB, CB), jnp.int32),
        pltpu.VMEM((IB, CB), jnp.int32),
        pltpu.VMEM((CB, H), jnp.float32),
        pltpu.VMEM((CB, H), jnp.float32),
        pltpu.VMEM((CB, H), jnp.float32),
        pltpu.VMEM((CB, H), jnp.float32),
        pltpu.VMEM_SHARED((N_PAD, H), jnp.float32),
    ] + [pltpu.SemaphoreType.DMA] * 8,
)

_gather = pl.kernel(
    _sc_gather, mesh=_mesh, compiler_params=_sc_params,
    out_type=jax.ShapeDtypeStruct((B, H), jnp.float32),
    scratch_types=[
        pltpu.VMEM((B // (NC * NS),), jnp.int32),
        pltpu.VMEM((B // (NC * NS), H), jnp.float32),
        pltpu.SemaphoreType.DMA,
    ],
)


_RB = 2000


def _dense_body(nparts, ncopies, refs):
    *sum_refs, cnt_ref, x_ref, wl_ref, bl_ref, wr_ref, g_ref, b_ref, o_ref = refs
    summed = sum_refs[0][...]
    for r in sum_refs[1:]:
        summed = summed + r[...]
    cnt = jnp.sum(cnt_ref[...], axis=1, keepdims=True)
    inv = 1.0 / jnp.maximum(cnt, 1.0)
    mean = summed * inv
    z = (jnp.dot(mean, wl_ref[...], preferred_element_type=jnp.float32)
         + bl_ref[...]
         + jnp.dot(x_ref[...], wr_ref[...], preferred_element_type=jnp.float32))
    mu = jnp.mean(z, axis=-1, keepdims=True)
    zc = z - mu
    var = jnp.mean(zc * zc, axis=-1, keepdims=True)
    y = zc * lax.rsqrt(var + 1e-5) * g_ref[...] + b_ref[...]
    out = jnp.maximum(y, 0.0)
    if ncopies == 1:
        o_ref[...] = out
    else:
        o_ref[...] = jnp.broadcast_to(out[None], (ncopies,) + out.shape)


def _dense(summed_list, cnt, x_dst, wl, bl, wr, g, b, ncopies=1):
    np_ = len(summed_list)
    row_spec = pl.BlockSpec((_RB, H), lambda i: (i, 0))
    full = pl.BlockSpec((H, H), lambda i: (0, 0))
    vec = pl.BlockSpec((1, H), lambda i: (0, 0))
    if ncopies == 1:
        out_spec = row_spec
        out_shape = jax.ShapeDtypeStruct((N, H), jnp.float32)
    else:
        out_spec = pl.BlockSpec((ncopies, _RB, H), lambda i: (0, i, 0))
        out_shape = jax.ShapeDtypeStruct((ncopies, N, H), jnp.float32)
    return pl.pallas_call(
        lambda *refs: _dense_body(np_, ncopies, refs),
        grid=(N // _RB,),
        in_specs=[row_spec] * np_ + [
            pl.BlockSpec((_RB, NS), lambda i: (i, 0)),
            row_spec, full, vec, full, vec, vec,
        ],
        out_specs=out_spec,
        out_shape=out_shape,
    )(*summed_list, cnt, x_dst, wl.reshape(H, H), bl.reshape(1, H),
      wr.reshape(H, H), g.reshape(1, H), b.reshape(1, H))


def _head_body(h_ref, w1_ref, b1_ref, w2_ref, b2_ref, o_ref):
    y = jnp.maximum(
        jnp.dot(h_ref[...], w1_ref[...], preferred_element_type=jnp.float32)
        + b1_ref[...], 0.0)
    w2 = w2_ref[...]
    o_ref[...] = jnp.sum(y * w2[:, 0][None, :], axis=1, keepdims=True) + b2_ref[...]


def _head(hrows, w1, b1, w2, b2):
    hb = 1024
    return pl.pallas_call(
        _head_body,
        grid=(B // hb,),
        in_specs=[
            pl.BlockSpec((hb, H), lambda i: (i, 0)),
            pl.BlockSpec((H, H // 2), lambda i: (0, 0)),
            pl.BlockSpec((1, H // 2), lambda i: (0, 0)),
            pl.BlockSpec((H // 2, 1), lambda i: (0, 0)),
            pl.BlockSpec((1, 1), lambda i: (0, 0)),
        ],
        out_specs=pl.BlockSpec((hb, 1), lambda i: (i, 0)),
        out_shape=jax.ShapeDtypeStruct((B, 1), jnp.float32),
    )(hrows, w1, b1.reshape(1, H // 2), w2, b2.reshape(1, 1))


def _pad_edges(ei, n_tiles, ept):
    share = E // n_tiles
    src = ei[0].reshape(n_tiles, share)
    dst = ei[1].reshape(n_tiles, share)
    pad = ept - share
    src = jnp.pad(src, ((0, 0), (0, pad)))
    dst = jnp.pad(dst, ((0, 0), (0, pad)), constant_values=PAD_DST)
    return (src.reshape(n_tiles, ept // CB, CB),
            dst.reshape(n_tiles, ept // CB, CB))


def kernel(edge_index_u2i, edge_index_i2u, target_ids, emb_user, emb_item,
           Wl_0_u2i, bl_0_u2i, Wr_0_u2i, Wl_0_i2u, bl_0_i2u, Wr_0_i2u,
           ln_g_0_user, ln_b_0_user, ln_g_0_item, ln_b_0_item,
           Wl_1_u2i, bl_1_u2i, Wr_1_u2i, Wl_1_i2u, bl_1_i2u, Wr_1_i2u,
           ln_g_1_user, ln_b_1_user, ln_g_1_item, ln_b_1_item,
           head_W1, head_b1, head_W2, head_b2):
    sA, dA = _pad_edges(edge_index_u2i, NS, EPT0)
    sB, dB = _pad_edges(edge_index_i2u, NS, EPT0)
    s1, d1 = _pad_edges(edge_index_u2i, NC * NS, EPT1)

    zeros = jnp.zeros((N_PAD, H), jnp.float32)
    emb_user2 = jnp.broadcast_to(emb_user[None], (4, N, H))
    emb_item2 = jnp.broadcast_to(emb_item[None], (4, N, H))

    sum_item0, sum_user0, cnt_raw = _seg0(sA, dA, sB, dB,
                                          emb_user2, emb_item2, zeros)
    cnt_parts = cnt_raw.reshape(NC, NS, N_PAD)
    cnt_item = cnt_parts[0].T
    cnt_user = cnt_parts[1].T

    x1_item = _dense([sum_item0], cnt_item, emb_item,
                     Wl_0_u2i, bl_0_u2i, Wr_0_u2i, ln_g_0_item, ln_b_0_item)
    x1_user = _dense([sum_user0], cnt_user, emb_user,
                     Wl_0_i2u, bl_0_i2u, Wr_0_i2u, ln_g_0_user, ln_b_0_user,
                     ncopies=4 * NC)

    parts = _seg1(s1, d1, x1_user, zeros)
    x2_item = _dense([parts[0], parts[1]], cnt_item, x1_item,
                     Wl_1_u2i, bl_1_u2i, Wr_1_u2i, ln_g_1_item, ln_b_1_item)

    hrows = _gather(x2_item, target_ids)
    out = _head(hrows, head_W1, head_b1, head_W2, head_b2)
    return out[:, 0]


# speedup vs baseline: 8.5939x; 1.0026x over previous
import jax
import jax.numpy as jnp
from jax import lax
from jax.experimental import pallas as pl
from jax.experimental.pallas import tpu as pltpu
from jax.experimental.pallas import tpu_sc as plsc

N = 10000
E = 320000
H = 128
B = 4096

NC, NS = 2, 16
CB = 64
IB = 16
DEPTH = 4
N_PAD = 10240
RPT = N_PAD // NS
PAD_DST = N + 100

EPT0 = 20480
EPT1 = 10240
NB0 = EPT0 // (IB * CB)
NB1 = EPT1 // (IB * CB)

_mesh = plsc.VectorSubcoreMesh(core_axis_name="c", subcore_axis_name="s")
_sc_params = pltpu.CompilerParams(needs_layout_passes=False)


def _run_block(b, src_e, dst_e, x_src, isrc, idst, rows, gs, ss, acc,
               cnt_priv, ones, tile):
    pltpu.sync_copy(src_e.at[tile, pl.ds(b * IB, IB)], isrc)
    pltpu.sync_copy(dst_e.at[tile, pl.ds(b * IB, IB)], idst)
    g = [None] * DEPTH
    s = [None] * DEPTH

    def swait_prev(q):
        @pl.when(b > 0)
        def _():
            pltpu.make_async_copy(rows[q], acc.at[idst.at[0]], ss[q]).wait()

    for j in range(DEPTH - 1):
        swait_prev(j)
        g[j] = pltpu.async_copy(x_src.at[isrc.at[j]], rows[j], gs[j])
    for r in range(IB):
        q = r % DEPTH
        g[q].wait()
        nxt = r + DEPTH - 1
        if nxt < IB:
            qn = nxt % DEPTH
            if nxt - DEPTH >= 0:
                s[qn].wait()
            elif nxt == DEPTH - 1:
                swait_prev(qn)
            g[qn] = pltpu.async_copy(x_src.at[isrc.at[nxt]], rows[qn], gs[qn])
        s[q] = pltpu.async_copy(rows[q], acc.at[idst.at[r]], ss[q], add=True)
        if cnt_priv is not None:
            for k in range(CB // 16):
                d = idst[r, pl.ds(k * 16, 16)]
                plsc.addupdate_scatter(cnt_priv, [d], ones)


def _accumulate_relation(src_e, dst_e, x_src, bufs, acc, tile, n_blocks,
                         cnt_priv):
    (isrc, idst, rows0, rows1, rows2, rows3,
     gs0, gs1, gs2, gs3, ss0, ss1, ss2, ss3) = bufs
    rows = (rows0, rows1, rows2, rows3)
    gs = (gs0, gs1, gs2, gs3)
    ss = (ss0, ss1, ss2, ss3)
    ones = jnp.full((16,), 1.0, jnp.float32)

    def block(b, carry):
        _run_block(b, src_e, dst_e, x_src, isrc, idst, rows, gs, ss, acc,
                   cnt_priv, ones, tile)
        return carry

    lax.fori_loop(0, n_blocks, block, 0)
    for q in range(DEPTH):
        pltpu.make_async_copy(rows[q], acc.at[idst.at[0]], ss[q]).wait()


def _sc_layer0(srcA, dstA, srcB, dstB, x_user, x_item, zeros,
               out_item, out_user, out_cnt,
               isrc, idst, rows0, rows1, rows2, rows3, cnt_priv, acc,
               gs0, gs1, gs2, gs3, ss0, ss1, ss2, ss3):
    c = lax.axis_index("c")
    s = lax.axis_index("s")
    wid = c * NS + s
    bufs = (isrc, idst, rows0, rows1, rows2, rows3,
            gs0, gs1, gs2, gs3, ss0, ss1, ss2, ss3)

    pltpu.sync_copy(zeros.at[pl.ds(s * RPT, RPT)], acc.at[pl.ds(s * RPT, RPT)])
    z16 = jnp.zeros((16,), jnp.float32)

    def zstep(t, carry):
        cnt_priv[pl.ds(t * 16, 16)] = z16
        return carry

    lax.fori_loop(0, N_PAD // 16, zstep, 0)
    plsc.subcore_barrier()

    cp = s & 3

    @pl.when(c == 0)
    def _():
        _accumulate_relation(srcA, dstA, x_user.at[cp], bufs, acc, s, NB0,
                             cnt_priv)

    @pl.when(c == 1)
    def _():
        _accumulate_relation(srcB, dstB, x_item.at[cp], bufs, acc, s, NB0,
                             cnt_priv)

    pltpu.sync_copy(cnt_priv, out_cnt.at[pl.ds(wid * N_PAD, N_PAD)])
    plsc.subcore_barrier()

    @pl.when(c == 0)
    def _():
        pltpu.sync_copy(acc.at[pl.ds(s * RPT, RPT)],
                        out_item.at[pl.ds(s * RPT, RPT)])

    @pl.when(c == 1)
    def _():
        pltpu.sync_copy(acc.at[pl.ds(s * RPT, RPT)],
                        out_user.at[pl.ds(s * RPT, RPT)])


def _sc_layer1(src1, dst1, x_user, zeros, out_part,
               isrc, idst, rows0, rows1, rows2, rows3, acc,
               gs0, gs1, gs2, gs3, ss0, ss1, ss2, ss3):
    c = lax.axis_index("c")
    s = lax.axis_index("s")
    wid = c * NS + s
    bufs = (isrc, idst, rows0, rows1, rows2, rows3,
            gs0, gs1, gs2, gs3, ss0, ss1, ss2, ss3)

    pltpu.sync_copy(zeros.at[pl.ds(s * RPT, RPT)], acc.at[pl.ds(s * RPT, RPT)])
    plsc.subcore_barrier()
    _accumulate_relation(src1, dst1, x_user.at[c * 4 + (s & 3)], bufs, acc,
                         wid, NB1, None)
    plsc.subcore_barrier()
    pltpu.sync_copy(acc.at[pl.ds(s * RPT, RPT)],
                    out_part.at[c, pl.ds(s * RPT, RPT)])


def _sc_gather(table, tids, out, idx_v, rows_v, sem):
    c = lax.axis_index("c")
    s = lax.axis_index("s")
    wid = c * NS + s
    bpw = B // (NC * NS)
    pltpu.sync_copy(tids.at[pl.ds(wid * bpw, bpw)], idx_v)
    pltpu.async_copy(table.at[idx_v], rows_v, sem).wait()
    pltpu.sync_copy(rows_v, out.at[pl.ds(wid * bpw, bpw)])


_seg0 = pl.kernel(
    _sc_layer0, mesh=_mesh, compiler_params=_sc_params,
    out_type=(
        jax.ShapeDtypeStruct((N_PAD, H), jnp.float32),
        jax.ShapeDtypeStruct((N_PAD, H), jnp.float32),
        jax.ShapeDtypeStruct((NC * NS * N_PAD,), jnp.float32),
    ),
    scratch_types=[
        pltpu.VMEM((IB, CB), jnp.int32),
        pltpu.VMEM((IB, CB), jnp.int32),
        pltpu.VMEM((CB, H), jnp.float32),
        pltpu.VMEM((CB, H), jnp.float32),
        pltpu.VMEM((CB, H), jnp.float32),
        pltpu.VMEM((CB, H), jnp.float32),
        pltpu.VMEM((N_PAD,), jnp.float32),
        pltpu.VMEM_SHARED((N_PAD, H), jnp.float32),
    ] + [pltpu.SemaphoreType.DMA] * 8,
)

_seg1 = pl.kernel(
    _sc_layer1, mesh=_mesh, compiler_params=_sc_params,
    out_type=jax.ShapeDtypeStruct((NC, N_PAD, H), jnp.float32),
    scratch_types=[
        pltpu.VMEM((IB, CB), jnp.int32),
        pltpu.VMEM((IB, CB), jnp.int32),
        pltpu.VMEM((CB, H), jnp.float32),
        pltpu.VMEM((CB, H), jnp.float32),
        pltpu.VMEM((CB, H), jnp.float32),
        pltpu.VMEM((CB, H), jnp.float32),
        pltpu.VMEM_SHARED((N_PAD, H), jnp.float32),
    ] + [pltpu.SemaphoreType.DMA] * 8,
)

_gather = pl.kernel(
    _sc_gather, mesh=_mesh, compiler_params=_sc_params,
    out_type=jax.ShapeDtypeStruct((B, H), jnp.float32),
    scratch_types=[
        pltpu.VMEM((B // (NC * NS),), jnp.int32),
        pltpu.VMEM((B // (NC * NS), H), jnp.float32),
        pltpu.SemaphoreType.DMA,
    ],
)


_RB = 2000


def _dense_body(nparts, ncopies, refs):
    *sum_refs, cnt_ref, x_ref, wl_ref, bl_ref, wr_ref, g_ref, b_ref, o_ref = refs
    summed = sum_refs[0][...]
    for r in sum_refs[1:]:
        summed = summed + r[...]
    cnt = jnp.sum(cnt_ref[...], axis=1, keepdims=True)
    inv = 1.0 / jnp.maximum(cnt, 1.0)
    mean = summed * inv
    z = (jnp.dot(mean, wl_ref[...], preferred_element_type=jnp.float32)
         + bl_ref[...]
         + jnp.dot(x_ref[...], wr_ref[...], preferred_element_type=jnp.float32))
    mu = jnp.mean(z, axis=-1, keepdims=True)
    zc = z - mu
    var = jnp.mean(zc * zc, axis=-1, keepdims=True)
    y = zc / jnp.sqrt(var + 1e-5) * g_ref[...] + b_ref[...]
    out = jnp.maximum(y, 0.0)
    if ncopies == 1:
        o_ref[...] = out
    else:
        o_ref[...] = jnp.broadcast_to(out[None], (ncopies,) + out.shape)


def _dense(summed_list, cnt, x_dst, wl, bl, wr, g, b, ncopies=1):
    np_ = len(summed_list)
    row_spec = pl.BlockSpec((_RB, H), lambda i: (i, 0))
    full = pl.BlockSpec((H, H), lambda i: (0, 0))
    vec = pl.BlockSpec((1, H), lambda i: (0, 0))
    if ncopies == 1:
        out_spec = row_spec
        out_shape = jax.ShapeDtypeStruct((N, H), jnp.float32)
    else:
        out_spec = pl.BlockSpec((ncopies, _RB, H), lambda i: (0, i, 0))
        out_shape = jax.ShapeDtypeStruct((ncopies, N, H), jnp.float32)
    return pl.pallas_call(
        lambda *refs: _dense_body(np_, ncopies, refs),
        grid=(N // _RB,),
        in_specs=[row_spec] * np_ + [
            pl.BlockSpec((_RB, NS), lambda i: (i, 0)),
            row_spec, full, vec, full, vec, vec,
        ],
        out_specs=out_spec,
        out_shape=out_shape,
    )(*summed_list, cnt, x_dst, wl.reshape(H, H), bl.reshape(1, H),
      wr.reshape(H, H), g.reshape(1, H), b.reshape(1, H))


def _head_body(h_ref, w1_ref, b1_ref, w2_ref, b2_ref, o_ref):
    y = jnp.maximum(
        jnp.dot(h_ref[...], w1_ref[...], preferred_element_type=jnp.float32)
        + b1_ref[...], 0.0)
    w2 = w2_ref[...]
    o_ref[...] = jnp.sum(y * w2[:, 0][None, :], axis=1, keepdims=True) + b2_ref[...]


def _head(hrows, w1, b1, w2, b2):
    hb = 1024
    return pl.pallas_call(
        _head_body,
        grid=(B // hb,),
        in_specs=[
            pl.BlockSpec((hb, H), lambda i: (i, 0)),
            pl.BlockSpec((H, H // 2), lambda i: (0, 0)),
            pl.BlockSpec((1, H // 2), lambda i: (0, 0)),
            pl.BlockSpec((H // 2, 1), lambda i: (0, 0)),
            pl.BlockSpec((1, 1), lambda i: (0, 0)),
        ],
        out_specs=pl.BlockSpec((hb, 1), lambda i: (i, 0)),
        out_shape=jax.ShapeDtypeStruct((B, 1), jnp.float32),
    )(hrows, w1, b1.reshape(1, H // 2), w2, b2.reshape(1, 1))


def _pad_edges(ei, n_tiles, ept):
    share = E // n_tiles
    src = ei[0].reshape(n_tiles, share)
    dst = ei[1].reshape(n_tiles, share)
    pad = ept - share
    src = jnp.pad(src, ((0, 0), (0, pad)))
    dst = jnp.pad(dst, ((0, 0), (0, pad)), constant_values=PAD_DST)
    return (src.reshape(n_tiles, ept // CB, CB),
            dst.reshape(n_tiles, ept // CB, CB))


def kernel(edge_index_u2i, edge_index_i2u, target_ids, emb_user, emb_item,
           Wl_0_u2i, bl_0_u2i, Wr_0_u2i, Wl_0_i2u, bl_0_i2u, Wr_0_i2u,
           ln_g_0_user, ln_b_0_user, ln_g_0_item, ln_b_0_item,
           Wl_1_u2i, bl_1_u2i, Wr_1_u2i, Wl_1_i2u, bl_1_i2u, Wr_1_i2u,
           ln_g_1_user, ln_b_1_user, ln_g_1_item, ln_b_1_item,
           head_W1, head_b1, head_W2, head_b2):
    sA, dA = _pad_edges(edge_index_u2i, NS, EPT0)
    sB, dB = _pad_edges(edge_index_i2u, NS, EPT0)
    s1, d1 = _pad_edges(edge_index_u2i, NC * NS, EPT1)

    zeros = jnp.zeros((N_PAD, H), jnp.float32)
    emb_user2 = jnp.broadcast_to(emb_user[None], (4, N, H))
    emb_item2 = jnp.broadcast_to(emb_item[None], (4, N, H))

    sum_item0, sum_user0, cnt_raw = _seg0(sA, dA, sB, dB,
                                          emb_user2, emb_item2, zeros)
    cnt_parts = cnt_raw.reshape(NC, NS, N_PAD)
    cnt_item = cnt_parts[0].T
    cnt_user = cnt_parts[1].T

    x1_item = _dense([sum_item0], cnt_item, emb_item,
                     Wl_0_u2i, bl_0_u2i, Wr_0_u2i, ln_g_0_item, ln_b_0_item)
    x1_user = _dense([sum_user0], cnt_user, emb_user,
                     Wl_0_i2u, bl_0_i2u, Wr_0_i2u, ln_g_0_user, ln_b_0_user,
                     ncopies=4 * NC)

    parts = _seg1(s1, d1, x1_user, zeros)
    x2_item = _dense([parts[0], parts[1]], cnt_item, x1_item,
                     Wl_1_u2i, bl_1_u2i, Wr_1_u2i, ln_g_1_item, ln_b_1_item)

    hrows = _gather(x2_item, target_ids)
    out = _head(hrows, head_W1, head_b1, head_W2, head_b2)
    return out[:, 0]
